# baseline (speedup 1.0000x reference)
import functools

import jax
import jax.numpy as jnp
from jax import lax
from jax.experimental import pallas as pl
from jax.experimental.pallas import tpu as pltpu
from jax.experimental.pallas import tpu_sc as plsc

N = 10000
M = 320000
D_IN = 768
D = 128
L = 2

NC = 2
NS = 16
NW = NC * NS
PER_W = M // NW
CHUNK = 400
N_CHUNKS = PER_W // CHUNK
ROWS_PER_TILE = N // NS
SCHUNK = 200
S_CHUNKS = PER_W // SCHUNK


def _lrelu(x):
    return jnp.where(x >= 0, x, 0.01 * x)


def _ln(x, g, b):
    mu = jnp.mean(x, axis=-1, keepdims=True)
    var = jnp.mean((x - mu) * (x - mu), axis=-1, keepdims=True)
    return (x - mu) * lax.rsqrt(var + 1e-5) * g + b


def _encoder(x, W, b, ln):
    BN = 1000

    def body(x_ref, w_ref, b_ref, ln_ref, o_ref):
        y = jnp.dot(x_ref[...], w_ref[...], preferred_element_type=jnp.float32)
        y = _lrelu(y + b_ref[...])
        o_ref[...] = _ln(y, ln_ref[0:1], ln_ref[1:2])

    return pl.pallas_call(
        body,
        grid=(N // BN,),
        in_specs=[
            pl.BlockSpec((BN, D_IN), lambda i: (i, 0)),
            pl.BlockSpec((D_IN, D), lambda i: (0, 0)),
            pl.BlockSpec((1, D), lambda i: (0, 0)),
            pl.BlockSpec((2, D), lambda i: (0, 0)),
        ],
        out_specs=pl.BlockSpec((BN, D), lambda i: (i, 0)),
        out_shape=jax.ShapeDtypeStruct((N, D), jnp.float32),
    )(x, W, b.reshape(1, D), ln)


def _edge(Hh, Ht, Eg, Ws, beu, bmf, bmb, lnr, rr=None, dir2=None, out_e=True):
    BE = 512
    first = rr is not None

    def body(*refs):
        if first:
            hh, ht, eg, rrr, d2, ws, b1, b2, b3, lnref = refs[:10]
            outs = refs[10:]
        else:
            hh, ht, eg, ws, b1, b2, b3, lnref = refs[:8]
            outs = refs[8:]
        Hhf = hh[...]
        Htf = ht[...]
        Ef = eg[...].astype(jnp.float32)
        if first:
            w = rrr[...]
            Ef = Ef + d2[0:1, :] * (1.0 - w) + d2[1:2, :] * w
        Hh_ = Hhf.astype(jnp.bfloat16)
        Ht_ = Htf.astype(jnp.bfloat16)
        Eb = Ef.astype(jnp.bfloat16)
        dot = functools.partial(jnp.dot, preferred_element_type=jnp.float32)
        T = dot(Hh_, ws[0]) + dot(Eb, ws[1]) + dot(Ht_, ws[2]) + b1[...]
        Enf = _ln(_lrelu(T) + Ef, lnref[0:1], lnref[1:2])
        Enb = Enf.astype(jnp.bfloat16)
        Pf = (Hhf * Enf).astype(jnp.bfloat16)
        Pb = (Htf * Enf).astype(jnp.bfloat16)
        mf = dot(Hh_, ws[3]) + dot(Enb, ws[4]) + dot(Pf, ws[5]) + b2[...]
        mb = dot(Ht_, ws[6]) + dot(Enb, ws[7]) + dot(Pb, ws[8]) + b3[...]
        if out_e:
            outs[0][...] = Enb
            outs[1][...] = mf
            outs[2][...] = mb
        else:
            outs[0][...] = mf
            outs[1][...] = mb

    eb = pl.BlockSpec((BE, D), lambda i: (i, 0))

    def cb(shape):
        return pl.BlockSpec(shape, lambda i, _s=shape: tuple(0 for _ in _s))

    in_specs = [eb, eb, eb]
    args = [Hh, Ht, Eg]
    if first:
        in_specs += [pl.BlockSpec((BE, 1), lambda i: (i, 0)), cb((2, D))]
        args += [rr, dir2]
    in_specs += [cb((9, D, D)), cb((1, D)), cb((1, D)), cb((1, D)), cb((2, D))]
    args += [Ws, beu.reshape(1, D), bmf.reshape(1, D), bmb.reshape(1, D), lnr]
    out_shape = [jax.ShapeDtypeStruct((M, D), jnp.float32),
                 jax.ShapeDtypeStruct((M, D), jnp.float32)]
    if out_e:
        out_shape = [jax.ShapeDtypeStruct((M, D), jnp.bfloat16)] + out_shape
    out = pl.pallas_call(
        body,
        grid=(M // BE,),
        in_specs=in_specs,
        out_specs=[eb] * len(out_shape),
        out_shape=out_shape,
    )(*args)
    return out


def _node(aggp, cntp3, H, lnr):
    BN = 1000

    def body(a_ref, c_ref, h_ref, ln_ref, o_ref):
        a = a_ref[0] + a_ref[1]
        cnt = c_ref[0] + c_ref[1]
        a = a / jnp.maximum(cnt, 1.0)
        x = _lrelu(a) + h_ref[...]
        o_ref[...] = _ln(x, ln_ref[0:1], ln_ref[1:2])

    return pl.pallas_call(
        body,
        grid=(N // BN,),
        in_specs=[
            pl.BlockSpec((2, BN, D), lambda i: (0, i, 0)),
            pl.BlockSpec((2, BN, 1), lambda i: (0, i, 0)),
            pl.BlockSpec((BN, D), lambda i: (i, 0)),
            pl.BlockSpec((2, D), lambda i: (0, 0)),
        ],
        out_specs=pl.BlockSpec((BN, D), lambda i: (i, 0)),
        out_shape=jax.ShapeDtypeStruct((N, D), jnp.float32),
    )(aggp, cntp3, H, lnr)


def _sc_gather(tables, idxs, dtype=jnp.float32):
    n = len(tables)
    mesh = plsc.VectorSubcoreMesh(core_axis_name="c", subcore_axis_name="s")
    out_type = tuple(jax.ShapeDtypeStruct((M, D), dtype) for _ in range(n))
    scratch = [
        pltpu.VMEM((CHUNK,), jnp.int32),
        pltpu.VMEM((CHUNK, D), dtype),
        pltpu.SemaphoreType.DMA,
    ]

    def body(*refs):
        tbl = refs[:n]
        idx = refs[n:2 * n]
        out = refs[2 * n:3 * n]
        idx_v, rows_v, sem = refs[3 * n:]
        wid = lax.axis_index("s") * NC + lax.axis_index("c")
        base = wid * PER_W

        def step(i, carry):
            off = base + i * CHUNK
            for t in range(n):
                pltpu.sync_copy(idx[t].at[pl.ds(off, CHUNK)], idx_v)
                pltpu.async_copy(tbl[t].at[idx_v], rows_v, sem).wait()
                pltpu.sync_copy(rows_v, out[t].at[pl.ds(off, CHUNK)])
            return carry

        lax.fori_loop(0, N_CHUNKS, step, 0)

    f = pl.kernel(body, out_type=out_type, mesh=mesh, scratch_types=scratch)
    return f(*tables, *idxs)


_WFULL = 632
_WLAST = N - (NS - 1) * _WFULL


def _sc_scatter(mf, mb, tails, heads, with_cnt):
    mesh = plsc.VectorSubcoreMesh(core_axis_name="c", subcore_axis_name="s")
    out_type = [jax.ShapeDtypeStruct((NC * N, D), jnp.float32)]
    scratch = [
        pltpu.VMEM_SHARED((N, D), jnp.float32),
        pltpu.VMEM((SCHUNK, D), jnp.float32),
        pltpu.VMEM((SCHUNK,), jnp.int32),
    ]
    zeros_blk = jnp.zeros((_WFULL, D), jnp.float32)
    args = [mf, mb, tails, heads, zeros_blk]
    if with_cnt:
        out_type.append(jax.ShapeDtypeStruct((NC * N,), jnp.float32))
        scratch += [pltpu.VMEM_SHARED((N,), jnp.float32),
                    pltpu.VMEM((SCHUNK,), jnp.float32),
                    pltpu.VMEM((N,), jnp.float32)]
        args += [jnp.zeros((N,), jnp.float32), jnp.ones((SCHUNK,), jnp.float32)]

    def body(*refs):
        if with_cnt:
            (mf_h, mb_h, t_h, h_h, zb_h, zn_h, on_h, agg_o, cnt_o,
             agg_s, buf_v, idx_v, cnt_s, ones_v, cnt_v) = refs
        else:
            mf_h, mb_h, t_h, h_h, zb_h, agg_o, agg_s, buf_v, idx_v = refs
        c = lax.axis_index("c")
        s = lax.axis_index("s")
        wid = s * NC + c
        r0 = pl.multiple_of(s * _WFULL, 8)

        @pl.when(s < NS - 1)
        def _():
            pltpu.sync_copy(zb_h, agg_s.at[pl.ds(r0, _WFULL)])

        @pl.when(s == NS - 1)
        def _():
            pltpu.sync_copy(zb_h.at[pl.ds(0, _WLAST)],
                            agg_s.at[pl.ds(r0, _WLAST)])

        if with_cnt:
            @pl.when(s == 0)
            def _():
                pltpu.sync_copy(zn_h, cnt_v)
                pltpu.sync_copy(cnt_v, cnt_s)
            pltpu.sync_copy(on_h, ones_v)
        plsc.subcore_barrier()
        base = wid * PER_W

        def step(i, carry):
            off = pl.multiple_of(base + i * SCHUNK, 8)
            pltpu.sync_copy(t_h.at[pl.ds(off, SCHUNK)], idx_v)
            pltpu.sync_copy(mf_h.at[pl.ds(off, SCHUNK)], buf_v)
            pltpu.sync_copy(buf_v, agg_s.at[idx_v], add=True)
            if with_cnt:
                pltpu.sync_copy(ones_v, cnt_s.at[idx_v], add=True)
            pltpu.sync_copy(h_h.at[pl.ds(off, SCHUNK)], idx_v)
            pltpu.sync_copy(mb_h.at[pl.ds(off, SCHUNK)], buf_v)
            pltpu.sync_copy(buf_v, agg_s.at[idx_v], add=True)
            if with_cnt:
                pltpu.sync_copy(ones_v, cnt_s.at[idx_v], add=True)
            return carry

        lax.fori_loop(0, S_CHUNKS, step, 0)
        plsc.subcore_barrier()
        o0 = pl.multiple_of(c * N + r0, 8)

        @pl.when(s < NS - 1)
        def _():
            pltpu.sync_copy(agg_s.at[pl.ds(r0, _WFULL)],
                            agg_o.at[pl.ds(o0, _WFULL)])

        @pl.when(s == NS - 1)
        def _():
            pltpu.sync_copy(agg_s.at[pl.ds(r0, _WLAST)],
                            agg_o.at[pl.ds(o0, _WLAST)])

        if with_cnt:
            @pl.when(s == 0)
            def _():
                pltpu.sync_copy(cnt_s, cnt_v)
                pltpu.sync_copy(cnt_v,
                                cnt_o.at[pl.ds(pl.multiple_of(c * N, 8), N)])

    f = pl.kernel(body, out_type=tuple(out_type), mesh=mesh,
                  scratch_types=scratch)
    res = f(*args)
    return res if with_cnt else res[0]


def _layer_weights(W_eu, b_eu, W_mf, b_mf, W_mb, b_mb, ln_eu, l):
    Wh, We, Wt = W_eu[l, :D], W_eu[l, D:2 * D], W_eu[l, 2 * D:]
    Afh = W_mf[l, :D] + W_mf[l, 2 * D:3 * D]
    Afe = W_mf[l, D:2 * D] + W_mf[l, 2 * D:3 * D]
    Afp = W_mf[l, 3 * D:]
    Abt = W_mb[l, :D] + W_mb[l, 2 * D:3 * D]
    Abe = W_mb[l, D:2 * D] + W_mb[l, 2 * D:3 * D]
    Abp = W_mb[l, 3 * D:]
    Ws = jnp.stack([Wh, We, Wt, Afh, Afe, Afp, Abt, Abe, Abp])
    return Ws.astype(jnp.bfloat16), b_eu[l], b_mf[l], b_mb[l], ln_eu[l]


def kernel(entity_feat, ht, r_tensor, r_relative, W_ent, b_ent, ln_ent,
           rel_emb, dir_emb, W_eu, b_eu, ln_eu, W_mf, b_mf, W_mb, b_mb,
           ln_mp):
    heads = ht[:, 0]
    tails = ht[:, 1]
    rr = r_relative.astype(jnp.float32).reshape(M, 1)

    H = _encoder(entity_feat, W_ent, b_ent, ln_ent)
    Hh, Ht, Erel = _sc_gather([H, H, rel_emb], [heads, tails, r_tensor])

    cnt3 = None
    E = None
    for l in range(L):
        Ws, beu, bmf, bmb, lnr = _layer_weights(
            W_eu, b_eu, W_mf, b_mf, W_mb, b_mb, ln_eu, l)
        if l == 0:
            E, mfm, mbm = _edge(Hh, Ht, Erel, Ws, beu, bmf, bmb, lnr,
                                rr=rr, dir2=dir_emb, out_e=True)
            aggp, cntp = _sc_scatter(mfm, mbm, tails, heads, True)
            cnt3 = cntp.reshape(NC, N, 1)
        else:
            mfm, mbm = _edge(Hh, Ht, E, Ws, beu, bmf, bmb, lnr, out_e=False)
            aggp = _sc_scatter(mfm, mbm, tails, heads, False)
        H = _node(aggp.reshape(NC, N, D), cnt3, H, ln_mp[l])
        if l == 0:
            Hh, Ht = _sc_gather([H, H], [heads, tails])
    return H


# speedup vs baseline: 1.3085x; 1.3085x over previous
import functools

import jax
import jax.numpy as jnp
from jax import lax
from jax.experimental import pallas as pl
from jax.experimental.pallas import tpu as pltpu
from jax.experimental.pallas import tpu_sc as plsc

N = 10000
M = 320000
D_IN = 768
D = 128
L = 2

NC = 2
NS = 16
NW = NC * NS
ROWS_PER_TILE = N // NS
_WFULL = 632
_WLAST = N - (NS - 1) * _WFULL
MH = M // 2


def _lrelu(x):
    return jnp.where(x >= 0, x, 0.01 * x)


def _ln(x, g, b):
    mu = jnp.mean(x, axis=-1, keepdims=True)
    var = jnp.mean((x - mu) * (x - mu), axis=-1, keepdims=True)
    return (x - mu) * lax.rsqrt(var + 1e-5) * g + b


def _chunk_for(per_w):
    for c in (400, 200, 100, 40):
        if per_w % c == 0 and c % 8 == 0:
            return c
    raise ValueError(per_w)


def _encoder(x, W, b, ln):
    BN = 1000

    def body(x_ref, w_ref, b_ref, ln_ref, o_ref):
        y = jnp.dot(x_ref[...], w_ref[...], preferred_element_type=jnp.float32)
        y = _lrelu(y + b_ref[...])
        o_ref[...] = _ln(y, ln_ref[0:1], ln_ref[1:2])

    return pl.pallas_call(
        body,
        grid=(N // BN,),
        in_specs=[
            pl.BlockSpec((BN, D_IN), lambda i: (i, 0)),
            pl.BlockSpec((D_IN, D), lambda i: (0, 0)),
            pl.BlockSpec((1, D), lambda i: (0, 0)),
            pl.BlockSpec((2, D), lambda i: (0, 0)),
        ],
        out_specs=pl.BlockSpec((BN, D), lambda i: (i, 0)),
        out_shape=jax.ShapeDtypeStruct((N, D), jnp.float32),
        name="tc_encoder",
    )(x, W, b.reshape(1, D), ln)


def _edge(Hh, Ht, Eg, Ws, beu, bmf, bmb, lnr, out_e, name):
    size = Hh.shape[0]
    BE = 640

    def body(hh, ht, eg, ws, b1, b2, b3, lnref, *outs):
        Hhf = hh[...]
        Htf = ht[...]
        Ef = eg[...].astype(jnp.float32)
        Hh_ = Hhf.astype(jnp.bfloat16)
        Ht_ = Htf.astype(jnp.bfloat16)
        Eb = Ef.astype(jnp.bfloat16)
        dot = functools.partial(jnp.dot, preferred_element_type=jnp.float32)
        T = dot(Hh_, ws[0]) + dot(Eb, ws[1]) + dot(Ht_, ws[2]) + b1[...]
        Enf = _ln(_lrelu(T) + Ef, lnref[0:1], lnref[1:2])
        Enb = Enf.astype(jnp.bfloat16)
        Pf = (Hhf * Enf).astype(jnp.bfloat16)
        Pb = (Htf * Enf).astype(jnp.bfloat16)
        mf = dot(Hh_, ws[3]) + dot(Enb, ws[4]) + dot(Pf, ws[5]) + b2[...]
        mb = dot(Ht_, ws[6]) + dot(Enb, ws[7]) + dot(Pb, ws[8]) + b3[...]
        if out_e:
            outs[0][...] = Enb
            outs[1][...] = mf
            outs[2][...] = mb
        else:
            outs[0][...] = mf
            outs[1][...] = mb

    eb = pl.BlockSpec((BE, D), lambda i: (i, 0))

    def cb(shape):
        return pl.BlockSpec(shape, lambda i, _s=shape: tuple(0 for _ in _s))

    in_specs = [eb, eb, eb,
                cb((9, D, D)), cb((1, D)), cb((1, D)), cb((1, D)), cb((2, D))]
    args = [Hh, Ht, Eg, Ws,
            beu.reshape(1, D), bmf.reshape(1, D), bmb.reshape(1, D), lnr]
    out_shape = [jax.ShapeDtypeStruct((size, D), jnp.float32),
                 jax.ShapeDtypeStruct((size, D), jnp.float32)]
    if out_e:
        out_shape = [jax.ShapeDtypeStruct((size, D), jnp.bfloat16)] + out_shape
    return pl.pallas_call(
        body,
        grid=(size // BE,),
        in_specs=in_specs,
        out_specs=[eb] * len(out_shape),
        out_shape=out_shape,
        name=name,
    )(*args)


def _node(aggps, cnt3s, H, lnr, name):
    BN = 1000
    na, nc = len(aggps), len(cnt3s)

    def body(*refs):
        aggs = refs[:na]
        cnts = refs[na:na + nc]
        h_ref, ln_ref, o_ref = refs[na + nc:]
        a = aggs[0][0] + aggs[0][1]
        for r in aggs[1:]:
            a = a + r[0] + r[1]
        cnt = cnts[0][0] + cnts[0][1]
        for r in cnts[1:]:
            cnt = cnt + r[0] + r[1]
        a = a / jnp.maximum(cnt, 1.0)
        x = _lrelu(a) + h_ref[...]
        o_ref[...] = _ln(x, ln_ref[0:1], ln_ref[1:2])

    return pl.pallas_call(
        body,
        grid=(N // BN,),
        in_specs=(
            [pl.BlockSpec((2, BN, D), lambda i: (0, i, 0))] * na
            + [pl.BlockSpec((2, BN, 1), lambda i: (0, i, 0))] * nc
            + [pl.BlockSpec((BN, D), lambda i: (i, 0)),
               pl.BlockSpec((2, D), lambda i: (0, 0))]
        ),
        out_specs=pl.BlockSpec((BN, D), lambda i: (i, 0)),
        out_shape=jax.ShapeDtypeStruct((N, D), jnp.float32),
        name=name,
    )(*aggps, *cnt3s, H, lnr)


def _sc_gather(tables, idxs, name):
    n = len(tables)
    size = idxs[0].shape[0]
    per_w = size // NW
    chunk = _chunk_for(per_w)
    n_chunks = per_w // chunk
    mesh = plsc.VectorSubcoreMesh(core_axis_name="c", subcore_axis_name="s")
    out_type = tuple(
        jax.ShapeDtypeStruct((size, D), jnp.float32) for _ in range(n))
    scratch = [
        pltpu.VMEM((chunk,), jnp.int32),
        pltpu.VMEM((chunk, D), jnp.float32),
        pltpu.SemaphoreType.DMA,
    ]

    def body(*refs):
        tbl = refs[:n]
        idx = refs[n:2 * n]
        out = refs[2 * n:3 * n]
        idx_v, rows_v, sem = refs[3 * n:]
        wid = lax.axis_index("s") * NC + lax.axis_index("c")
        base = wid * per_w

        def step(i, carry):
            off = pl.multiple_of(base + i * chunk, 8)
            for t in range(n):
                pltpu.sync_copy(idx[t].at[pl.ds(off, chunk)], idx_v)
                pltpu.async_copy(tbl[t].at[idx_v], rows_v, sem).wait()
                pltpu.sync_copy(rows_v, out[t].at[pl.ds(off, chunk)])
            return carry

        lax.fori_loop(0, n_chunks, step, 0)

    f = pl.kernel(body, out_type=out_type, mesh=mesh, scratch_types=scratch,
                  name=name)
    return f(*tables, *idxs)


def _sc_scatter(mf, mb, tails, heads, with_cnt, name):
    size = tails.shape[0]
    per_w = size // NW
    chunk = 200
    n_chunks = per_w // chunk
    mesh = plsc.VectorSubcoreMesh(core_axis_name="c", subcore_axis_name="s")
    out_type = [jax.ShapeDtypeStruct((NC * N, D), jnp.float32)]
    scratch = [
        pltpu.VMEM_SHARED((N, D), jnp.float32),
        pltpu.VMEM((chunk, D), jnp.float32),
        pltpu.VMEM((chunk,), jnp.int32),
    ]
    zeros_blk = jnp.zeros((_WFULL, D), jnp.float32)
    args = [mf, mb, tails, heads, zeros_blk]
    if with_cnt:
        out_type.append(jax.ShapeDtypeStruct((NC * N,), jnp.float32))
        scratch += [pltpu.VMEM_SHARED((N,), jnp.float32),
                    pltpu.VMEM((chunk,), jnp.float32),
                    pltpu.VMEM((N,), jnp.float32)]
        args += [jnp.zeros((N,), jnp.float32), jnp.ones((chunk,), jnp.float32)]

    def body(*refs):
        if with_cnt:
            (mf_h, mb_h, t_h, h_h, zb_h, zn_h, on_h, agg_o, cnt_o,
             agg_s, buf_v, idx_v, cnt_s, ones_v, cnt_v) = refs
        else:
            mf_h, mb_h, t_h, h_h, zb_h, agg_o, agg_s, buf_v, idx_v = refs
        c = lax.axis_index("c")
        s = lax.axis_index("s")
        wid = s * NC + c
        r0 = pl.multiple_of(s * _WFULL, 8)

        @pl.when(s < NS - 1)
        def _():
            pltpu.sync_copy(zb_h, agg_s.at[pl.ds(r0, _WFULL)])

        @pl.when(s == NS - 1)
        def _():
            pltpu.sync_copy(zb_h.at[pl.ds(0, _WLAST)],
                            agg_s.at[pl.ds(r0, _WLAST)])

        if with_cnt:
            @pl.when(s == 0)
            def _():
                pltpu.sync_copy(zn_h, cnt_v)
                pltpu.sync_copy(cnt_v, cnt_s)
            pltpu.sync_copy(on_h, ones_v)
        plsc.subcore_barrier()
        base = wid * per_w

        def step(i, carry):
            off = pl.multiple_of(base + i * chunk, 8)
            pltpu.sync_copy(t_h.at[pl.ds(off, chunk)], idx_v)
            pltpu.sync_copy(mf_h.at[pl.ds(off, chunk)], buf_v)
            pltpu.sync_copy(buf_v, agg_s.at[idx_v], add=True)
            if with_cnt:
                pltpu.sync_copy(ones_v, cnt_s.at[idx_v], add=True)
            pltpu.sync_copy(h_h.at[pl.ds(off, chunk)], idx_v)
            pltpu.sync_copy(mb_h.at[pl.ds(off, chunk)], buf_v)
            pltpu.sync_copy(buf_v, agg_s.at[idx_v], add=True)
            if with_cnt:
                pltpu.sync_copy(ones_v, cnt_s.at[idx_v], add=True)
            return carry

        lax.fori_loop(0, n_chunks, step, 0)
        plsc.subcore_barrier()
        o0 = pl.multiple_of(c * N + r0, 8)

        @pl.when(s < NS - 1)
        def _():
            pltpu.sync_copy(agg_s.at[pl.ds(r0, _WFULL)],
                            agg_o.at[pl.ds(o0, _WFULL)])

        @pl.when(s == NS - 1)
        def _():
            pltpu.sync_copy(agg_s.at[pl.ds(r0, _WLAST)],
                            agg_o.at[pl.ds(o0, _WLAST)])

        if with_cnt:
            @pl.when(s == 0)
            def _():
                pltpu.sync_copy(cnt_s, cnt_v)
                pltpu.sync_copy(cnt_v,
                                cnt_o.at[pl.ds(pl.multiple_of(c * N, 8), N)])

    f = pl.kernel(body, out_type=tuple(out_type), mesh=mesh,
                  scratch_types=scratch, name=name)
    res = f(*args)
    return res if with_cnt else res[0]


def _layer_weights(W_eu, b_eu, W_mf, b_mf, W_mb, b_mb, ln_eu, l):
    Wh, We, Wt = W_eu[l, :D], W_eu[l, D:2 * D], W_eu[l, 2 * D:]
    Afh = W_mf[l, :D] + W_mf[l, 2 * D:3 * D]
    Afe = W_mf[l, D:2 * D] + W_mf[l, 2 * D:3 * D]
    Afp = W_mf[l, 3 * D:]
    Abt = W_mb[l, :D] + W_mb[l, 2 * D:3 * D]
    Abe = W_mb[l, D:2 * D] + W_mb[l, 2 * D:3 * D]
    Abp = W_mb[l, 3 * D:]
    Ws = jnp.stack([Wh, We, Wt, Afh, Afe, Afp, Abt, Abe, Abp])
    return Ws.astype(jnp.bfloat16), b_eu[l], b_mf[l], b_mb[l], ln_eu[l]


def kernel(entity_feat, ht, r_tensor, r_relative, W_ent, b_ent, ln_ent,
           rel_emb, dir_emb, W_eu, b_eu, ln_eu, W_mf, b_mf, W_mb, b_mb,
           ln_mp):
    heads = ht[:, 0]
    tails = ht[:, 1]
    T2 = (rel_emb[:, None, :] + dir_emb[None, :, :]).reshape(-1, D)
    idx_e = r_tensor * 2 + r_relative
    halves = ((0, MH), (MH, M))

    H = _encoder(entity_feat, W_ent, b_ent, ln_ent)
    Eh = [_sc_gather([T2], [idx_e[lo:hi]], f"sc_gE{i}")[0]
          for i, (lo, hi) in enumerate(halves)]
    gh = [_sc_gather([H, H], [heads[lo:hi], tails[lo:hi]], f"sc_gH0_{i}")
          for i, (lo, hi) in enumerate(halves)]

    cnt3s = None
    for l in range(L):
        Ws, beu, bmf, bmb, lnr = _layer_weights(
            W_eu, b_eu, W_mf, b_mf, W_mb, b_mb, ln_eu, l)
        first = l == 0
        eouts = [_edge(gh[i][0], gh[i][1], Eh[i], Ws, beu, bmf, bmb, lnr,
                       out_e=first, name=f"tc_edge{l}_{i}")
                 for i in range(2)]
        if first:
            Eh = [eo[0] for eo in eouts]
            scs = [_sc_scatter(eo[1], eo[2], tails[lo:hi], heads[lo:hi],
                               True, f"sc_scat{l}_{i}")
                   for i, (eo, (lo, hi)) in enumerate(zip(eouts, halves))]
            aggps = [s[0].reshape(NC, N, D) for s in scs]
            cnt3s = [s[1].reshape(NC, N, 1) for s in scs]
        else:
            scs = [_sc_scatter(eo[0], eo[1], tails[lo:hi], heads[lo:hi],
                               False, f"sc_scat{l}_{i}")
                   for i, (eo, (lo, hi)) in enumerate(zip(eouts, halves))]
            aggps = [s.reshape(NC, N, D) for s in scs]
        H = _node(aggps, cnt3s, H, ln_mp[l], f"tc_node{l}")
        if first:
            gh = [_sc_gather([H, H], [heads[lo:hi], tails[lo:hi]],
                             f"sc_gH1_{i}")
                  for i, (lo, hi) in enumerate(halves)]
    return H


# speedup vs baseline: 1.4423x; 1.1022x over previous
import functools

import jax
import jax.numpy as jnp
from jax import lax
from jax.experimental import pallas as pl
from jax.experimental.pallas import tpu as pltpu
from jax.experimental.pallas import tpu_sc as plsc

N = 10000
M = 320000
D_IN = 768
D = 128
L = 2

NC = 2
NS = 16
NW = NC * NS
ROWS_PER_TILE = N // NS
_WFULL = 632
_WLAST = N - (NS - 1) * _WFULL
NSPLIT = 5
MCH = M // NSPLIT


def _lrelu(x):
    return jnp.where(x >= 0, x, 0.01 * x)


def _ln(x, g, b):
    mu = jnp.mean(x, axis=-1, keepdims=True)
    var = jnp.mean((x - mu) * (x - mu), axis=-1, keepdims=True)
    return (x - mu) * lax.rsqrt(var + 1e-5) * g + b


def _chunk_for(per_w):
    for c in (400, 200, 100, 40):
        if per_w % c == 0 and c % 8 == 0:
            return c
    raise ValueError(per_w)


def _encoder(x, W, b, ln):
    BN = 1000

    def body(x_ref, w_ref, b_ref, ln_ref, o_ref):
        y = jnp.dot(x_ref[...], w_ref[...], preferred_element_type=jnp.float32)
        y = _lrelu(y + b_ref[...])
        o_ref[...] = _ln(y, ln_ref[0:1], ln_ref[1:2])

    return pl.pallas_call(
        body,
        grid=(N // BN,),
        in_specs=[
            pl.BlockSpec((BN, D_IN), lambda i: (i, 0)),
            pl.BlockSpec((D_IN, D), lambda i: (0, 0)),
            pl.BlockSpec((1, D), lambda i: (0, 0)),
            pl.BlockSpec((2, D), lambda i: (0, 0)),
        ],
        out_specs=pl.BlockSpec((BN, D), lambda i: (i, 0)),
        out_shape=jax.ShapeDtypeStruct((N, D), jnp.float32),
        name="tc_encoder",
    )(x, W, b.reshape(1, D), ln)


def _edge(Hh, Ht, Eg, Ws, beu, bmf, bmb, lnr, out_e, name):
    size = Hh.shape[0]
    BE = 640

    def body(hh, ht, eg, ws, b1, b2, b3, lnref, *outs):
        Hhf = hh[...]
        Htf = ht[...]
        Ef = eg[...].astype(jnp.float32)
        Hh_ = Hhf.astype(jnp.bfloat16)
        Ht_ = Htf.astype(jnp.bfloat16)
        Eb = Ef.astype(jnp.bfloat16)
        dot = functools.partial(jnp.dot, preferred_element_type=jnp.float32)
        T = dot(Hh_, ws[0]) + dot(Eb, ws[1]) + dot(Ht_, ws[2]) + b1[...]
        Enf = _ln(_lrelu(T) + Ef, lnref[0:1], lnref[1:2])
        Enb = Enf.astype(jnp.bfloat16)
        Pf = (Hhf * Enf).astype(jnp.bfloat16)
        Pb = (Htf * Enf).astype(jnp.bfloat16)
        mf = dot(Hh_, ws[3]) + dot(Enb, ws[4]) + dot(Pf, ws[5]) + b2[...]
        mb = dot(Ht_, ws[6]) + dot(Enb, ws[7]) + dot(Pb, ws[8]) + b3[...]
        if out_e:
            outs[0][...] = Enb
            outs[1][...] = mf
            outs[2][...] = mb
        else:
            outs[0][...] = mf
            outs[1][...] = mb

    eb = pl.BlockSpec((BE, D), lambda i: (i, 0))

    def cb(shape):
        return pl.BlockSpec(shape, lambda i, _s=shape: tuple(0 for _ in _s))

    in_specs = [eb, eb, eb,
                cb((9, D, D)), cb((1, D)), cb((1, D)), cb((1, D)), cb((2, D))]
    args = [Hh, Ht, Eg, Ws,
            beu.reshape(1, D), bmf.reshape(1, D), bmb.reshape(1, D), lnr]
    out_shape = [jax.ShapeDtypeStruct((size, D), jnp.float32),
                 jax.ShapeDtypeStruct((size, D), jnp.float32)]
    if out_e:
        out_shape = [jax.ShapeDtypeStruct((size, D), jnp.bfloat16)] + out_shape
    return pl.pallas_call(
        body,
        grid=(size // BE,),
        in_specs=in_specs,
        out_specs=[eb] * len(out_shape),
        out_shape=out_shape,
        name=name,
    )(*args)


def _node(aggps, cnt3s, H, lnr, name):
    BN = 1000
    na, nc = len(aggps), len(cnt3s)

    def body(*refs):
        aggs = refs[:na]
        cnts = refs[na:na + nc]
        h_ref, ln_ref, o_ref = refs[na + nc:]
        a = aggs[0][0] + aggs[0][1]
        for r in aggs[1:]:
            a = a + r[0] + r[1]
        cnt = cnts[0][0] + cnts[0][1]
        for r in cnts[1:]:
            cnt = cnt + r[0] + r[1]
        a = a / jnp.maximum(cnt, 1.0)
        x = _lrelu(a) + h_ref[...]
        o_ref[...] = _ln(x, ln_ref[0:1], ln_ref[1:2])

    return pl.pallas_call(
        body,
        grid=(N // BN,),
        in_specs=(
            [pl.BlockSpec((2, BN, D), lambda i: (0, i, 0))] * na
            + [pl.BlockSpec((2, BN, 1), lambda i: (0, i, 0))] * nc
            + [pl.BlockSpec((BN, D), lambda i: (i, 0)),
               pl.BlockSpec((2, D), lambda i: (0, 0))]
        ),
        out_specs=pl.BlockSpec((BN, D), lambda i: (i, 0)),
        out_shape=jax.ShapeDtypeStruct((N, D), jnp.float32),
        name=name,
    )(*aggps, *cnt3s, H, lnr)


def _sc_gather(tables, idxs, name):
    n = len(tables)
    size = idxs[0].shape[0]
    per_w = size // NW
    chunk = _chunk_for(per_w)
    n_chunks = per_w // chunk
    mesh = plsc.VectorSubcoreMesh(core_axis_name="c", subcore_axis_name="s")
    out_type = tuple(
        jax.ShapeDtypeStruct((size, D), jnp.float32) for _ in range(n))
    scratch = [
        pltpu.VMEM((chunk,), jnp.int32),
        pltpu.VMEM((chunk, D), jnp.float32),
        pltpu.SemaphoreType.DMA,
    ]

    def body(*refs):
        tbl = refs[:n]
        idx = refs[n:2 * n]
        out = refs[2 * n:3 * n]
        idx_v, rows_v, sem = refs[3 * n:]
        wid = lax.axis_index("s") * NC + lax.axis_index("c")
        base = wid * per_w

        def step(i, carry):
            off = pl.multiple_of(base + i * chunk, 8)
            for t in range(n):
                pltpu.sync_copy(idx[t].at[pl.ds(off, chunk)], idx_v)
                pltpu.async_copy(tbl[t].at[idx_v], rows_v, sem).wait()
                pltpu.sync_copy(rows_v, out[t].at[pl.ds(off, chunk)])
            return carry

        lax.fori_loop(0, n_chunks, step, 0)

    f = pl.kernel(body, out_type=out_type, mesh=mesh, scratch_types=scratch,
                  name=name)
    return f(*tables, *idxs)


def _sc_scatter(mf, mb, tails, heads, with_cnt, name):
    size = tails.shape[0]
    per_w = size // NW
    chunk = 200
    n_chunks = per_w // chunk
    mesh = plsc.VectorSubcoreMesh(core_axis_name="c", subcore_axis_name="s")
    out_type = [jax.ShapeDtypeStruct((NC * N, D), jnp.float32)]
    scratch = [
        pltpu.VMEM_SHARED((N, D), jnp.float32),
        pltpu.VMEM((chunk, D), jnp.float32),
        pltpu.VMEM((chunk,), jnp.int32),
    ]
    zeros_blk = jnp.zeros((_WFULL, D), jnp.float32)
    args = [mf, mb, tails, heads, zeros_blk]
    if with_cnt:
        out_type.append(jax.ShapeDtypeStruct((NC * N,), jnp.float32))
        scratch += [pltpu.VMEM_SHARED((N,), jnp.float32),
                    pltpu.VMEM((chunk,), jnp.float32),
                    pltpu.VMEM((N,), jnp.float32)]
        args += [jnp.zeros((N,), jnp.float32), jnp.ones((chunk,), jnp.float32)]

    def body(*refs):
        if with_cnt:
            (mf_h, mb_h, t_h, h_h, zb_h, zn_h, on_h, agg_o, cnt_o,
             agg_s, buf_v, idx_v, cnt_s, ones_v, cnt_v) = refs
        else:
            mf_h, mb_h, t_h, h_h, zb_h, agg_o, agg_s, buf_v, idx_v = refs
        c = lax.axis_index("c")
        s = lax.axis_index("s")
        wid = s * NC + c
        r0 = pl.multiple_of(s * _WFULL, 8)

        @pl.when(s < NS - 1)
        def _():
            pltpu.sync_copy(zb_h, agg_s.at[pl.ds(r0, _WFULL)])

        @pl.when(s == NS - 1)
        def _():
            pltpu.sync_copy(zb_h.at[pl.ds(0, _WLAST)],
                            agg_s.at[pl.ds(r0, _WLAST)])

        if with_cnt:
            @pl.when(s == 0)
            def _():
                pltpu.sync_copy(zn_h, cnt_v)
                pltpu.sync_copy(cnt_v, cnt_s)
            pltpu.sync_copy(on_h, ones_v)
        plsc.subcore_barrier()
        base = wid * per_w

        def step(i, carry):
            off = pl.multiple_of(base + i * chunk, 8)
            pltpu.sync_copy(t_h.at[pl.ds(off, chunk)], idx_v)
            pltpu.sync_copy(mf_h.at[pl.ds(off, chunk)], buf_v)
            pltpu.sync_copy(buf_v, agg_s.at[idx_v], add=True)
            if with_cnt:
                pltpu.sync_copy(ones_v, cnt_s.at[idx_v], add=True)
            pltpu.sync_copy(h_h.at[pl.ds(off, chunk)], idx_v)
            pltpu.sync_copy(mb_h.at[pl.ds(off, chunk)], buf_v)
            pltpu.sync_copy(buf_v, agg_s.at[idx_v], add=True)
            if with_cnt:
                pltpu.sync_copy(ones_v, cnt_s.at[idx_v], add=True)
            return carry

        lax.fori_loop(0, n_chunks, step, 0)
        plsc.subcore_barrier()
        o0 = pl.multiple_of(c * N + r0, 8)

        @pl.when(s < NS - 1)
        def _():
            pltpu.sync_copy(agg_s.at[pl.ds(r0, _WFULL)],
                            agg_o.at[pl.ds(o0, _WFULL)])

        @pl.when(s == NS - 1)
        def _():
            pltpu.sync_copy(agg_s.at[pl.ds(r0, _WLAST)],
                            agg_o.at[pl.ds(o0, _WLAST)])

        if with_cnt:
            @pl.when(s == 0)
            def _():
                pltpu.sync_copy(cnt_s, cnt_v)
                pltpu.sync_copy(cnt_v,
                                cnt_o.at[pl.ds(pl.multiple_of(c * N, 8), N)])

    f = pl.kernel(body, out_type=tuple(out_type), mesh=mesh,
                  scratch_types=scratch, name=name)
    res = f(*args)
    return res if with_cnt else res[0]


def _layer_weights(W_eu, b_eu, W_mf, b_mf, W_mb, b_mb, ln_eu, l):
    Wh, We, Wt = W_eu[l, :D], W_eu[l, D:2 * D], W_eu[l, 2 * D:]
    Afh = W_mf[l, :D] + W_mf[l, 2 * D:3 * D]
    Afe = W_mf[l, D:2 * D] + W_mf[l, 2 * D:3 * D]
    Afp = W_mf[l, 3 * D:]
    Abt = W_mb[l, :D] + W_mb[l, 2 * D:3 * D]
    Abe = W_mb[l, D:2 * D] + W_mb[l, 2 * D:3 * D]
    Abp = W_mb[l, 3 * D:]
    Ws = jnp.stack([Wh, We, Wt, Afh, Afe, Afp, Abt, Abe, Abp])
    return Ws.astype(jnp.bfloat16), b_eu[l], b_mf[l], b_mb[l], ln_eu[l]


def kernel(entity_feat, ht, r_tensor, r_relative, W_ent, b_ent, ln_ent,
           rel_emb, dir_emb, W_eu, b_eu, ln_eu, W_mf, b_mf, W_mb, b_mb,
           ln_mp):
    heads = ht[:, 0]
    tails = ht[:, 1]
    T2 = (rel_emb[:, None, :] + dir_emb[None, :, :]).reshape(-1, D)
    idx_e = r_tensor * 2 + r_relative
    halves = tuple((i * MCH, (i + 1) * MCH) for i in range(NSPLIT))

    H = _encoder(entity_feat, W_ent, b_ent, ln_ent)
    Eh = [_sc_gather([T2], [idx_e[lo:hi]], f"sc_gE{i}")[0]
          for i, (lo, hi) in enumerate(halves)]
    gh = [_sc_gather([H, H], [heads[lo:hi], tails[lo:hi]], f"sc_gH0_{i}")
          for i, (lo, hi) in enumerate(halves)]

    cnt3s = None
    for l in range(L):
        Ws, beu, bmf, bmb, lnr = _layer_weights(
            W_eu, b_eu, W_mf, b_mf, W_mb, b_mb, ln_eu, l)
        first = l == 0
        eouts = [_edge(gh[i][0], gh[i][1], Eh[i], Ws, beu, bmf, bmb, lnr,
                       out_e=first, name=f"tc_edge{l}_{i}")
                 for i in range(NSPLIT)]
        if first:
            Eh = [eo[0] for eo in eouts]
            scs = [_sc_scatter(eo[1], eo[2], tails[lo:hi], heads[lo:hi],
                               True, f"sc_scat{l}_{i}")
                   for i, (eo, (lo, hi)) in enumerate(zip(eouts, halves))]
            aggps = [s[0].reshape(NC, N, D) for s in scs]
            cnt3s = [s[1].reshape(NC, N, 1) for s in scs]
        else:
            scs = [_sc_scatter(eo[0], eo[1], tails[lo:hi], heads[lo:hi],
                               False, f"sc_scat{l}_{i}")
                   for i, (eo, (lo, hi)) in enumerate(zip(eouts, halves))]
            aggps = [s.reshape(NC, N, D) for s in scs]
        H = _node(aggps, cnt3s, H, ln_mp[l], f"tc_node{l}")
        if first:
            gh = [_sc_gather([H, H], [heads[lo:hi], tails[lo:hi]],
                             f"sc_gH1_{i}")
                  for i, (lo, hi) in enumerate(halves)]
    return H


# speedup vs baseline: 1.4874x; 1.0313x over previous
import functools

import jax
import jax.numpy as jnp
from jax import lax
from jax.experimental import pallas as pl
from jax.experimental.pallas import tpu as pltpu
from jax.experimental.pallas import tpu_sc as plsc

N = 10000
M = 320000
D_IN = 768
D = 128
L = 2

NC = 2
NS = 16
NW = NC * NS
ROWS_PER_TILE = N // NS
_WFULL = 632
_WLAST = N - (NS - 1) * _WFULL
NSPLIT = 5
MCH = M // NSPLIT


def _lrelu(x):
    return jnp.where(x >= 0, x, 0.01 * x)


def _ln(x, g, b):
    mu = jnp.mean(x, axis=-1, keepdims=True)
    var = jnp.mean((x - mu) * (x - mu), axis=-1, keepdims=True)
    return (x - mu) * lax.rsqrt(var + 1e-5) * g + b


def _chunk_for(per_w):
    for c in (400, 200, 100, 40):
        if per_w % c == 0 and c % 8 == 0:
            return c
    raise ValueError(per_w)


def _encoder(x, W, b, ln):
    BN = 1000

    def body(x_ref, w_ref, b_ref, ln_ref, o_ref):
        y = jnp.dot(x_ref[...], w_ref[...], preferred_element_type=jnp.float32)
        y = _lrelu(y + b_ref[...])
        o_ref[...] = _ln(y, ln_ref[0:1], ln_ref[1:2])

    return pl.pallas_call(
        body,
        grid=(N // BN,),
        in_specs=[
            pl.BlockSpec((BN, D_IN), lambda i: (i, 0)),
            pl.BlockSpec((D_IN, D), lambda i: (0, 0)),
            pl.BlockSpec((1, D), lambda i: (0, 0)),
            pl.BlockSpec((2, D), lambda i: (0, 0)),
        ],
        out_specs=pl.BlockSpec((BN, D), lambda i: (i, 0)),
        out_shape=jax.ShapeDtypeStruct((N, D), jnp.float32),
        name="tc_encoder",
    )(x, W, b.reshape(1, D), ln)


def _edge(Hh, Ht, Eg, Ws, beu, bmf, bmb, lnr, out_e, name):
    size = Hh.shape[0]
    BE = 640

    def body(hh, ht, eg, ws, b1, b2, b3, lnref, *outs):
        Hhf = hh[...]
        Htf = ht[...]
        Ef = eg[...].astype(jnp.float32)
        Hh_ = Hhf.astype(jnp.bfloat16)
        Ht_ = Htf.astype(jnp.bfloat16)
        Eb = Ef.astype(jnp.bfloat16)
        dot = functools.partial(jnp.dot, preferred_element_type=jnp.float32)
        T = dot(Hh_, ws[0]) + dot(Eb, ws[1]) + dot(Ht_, ws[2]) + b1[...]
        Enf = _ln(_lrelu(T) + Ef, lnref[0:1], lnref[1:2])
        Enb = Enf.astype(jnp.bfloat16)
        Pf = (Hhf * Enf).astype(jnp.bfloat16)
        Pb = (Htf * Enf).astype(jnp.bfloat16)
        mf = dot(Hh_, ws[3]) + dot(Enb, ws[4]) + dot(Pf, ws[5]) + b2[...]
        mb = dot(Ht_, ws[6]) + dot(Enb, ws[7]) + dot(Pb, ws[8]) + b3[...]
        if out_e:
            outs[0][...] = Enb
            outs[1][...] = mf
            outs[2][...] = mb
        else:
            outs[0][...] = mf
            outs[1][...] = mb

    eb = pl.BlockSpec((BE, D), lambda i: (i, 0))

    def cb(shape):
        return pl.BlockSpec(shape, lambda i, _s=shape: tuple(0 for _ in _s))

    in_specs = [eb, eb, eb,
                cb((9, D, D)), cb((1, D)), cb((1, D)), cb((1, D)), cb((2, D))]
    args = [Hh, Ht, Eg, Ws,
            beu.reshape(1, D), bmf.reshape(1, D), bmb.reshape(1, D), lnr]
    out_shape = [jax.ShapeDtypeStruct((size, D), jnp.float32),
                 jax.ShapeDtypeStruct((size, D), jnp.float32)]
    if out_e:
        out_shape = [jax.ShapeDtypeStruct((size, D), jnp.bfloat16)] + out_shape
    return pl.pallas_call(
        body,
        grid=(size // BE,),
        in_specs=in_specs,
        out_specs=[eb] * len(out_shape),
        out_shape=out_shape,
        name=name,
    )(*args)


def _node(aggps, cnt3s, H, lnr, name):
    BN = 1000
    na, nc = len(aggps), len(cnt3s)

    def body(*refs):
        aggs = refs[:na]
        cnts = refs[na:na + nc]
        h_ref, ln_ref, o_ref = refs[na + nc:]
        a = aggs[0][0] + aggs[0][1]
        for r in aggs[1:]:
            a = a + r[0] + r[1]
        cnt = cnts[0][0] + cnts[0][1]
        for r in cnts[1:]:
            cnt = cnt + r[0] + r[1]
        a = a / jnp.maximum(cnt, 1.0)
        x = _lrelu(a) + h_ref[...]
        o_ref[...] = _ln(x, ln_ref[0:1], ln_ref[1:2])

    return pl.pallas_call(
        body,
        grid=(N // BN,),
        in_specs=(
            [pl.BlockSpec((2, BN, D), lambda i: (0, i, 0))] * na
            + [pl.BlockSpec((2, BN, 1), lambda i: (0, i, 0))] * nc
            + [pl.BlockSpec((BN, D), lambda i: (i, 0)),
               pl.BlockSpec((2, D), lambda i: (0, 0))]
        ),
        out_specs=pl.BlockSpec((BN, D), lambda i: (i, 0)),
        out_shape=jax.ShapeDtypeStruct((N, D), jnp.float32),
        name=name,
    )(*aggps, *cnt3s, H, lnr)


def _sc_gather(tables, idxs, name):
    n = len(tables)
    size = idxs[0].shape[0]
    per_w = size // NW
    chunk = 200 if n >= 3 else _chunk_for(per_w)
    n_chunks = per_w // chunk
    mesh = plsc.VectorSubcoreMesh(core_axis_name="c", subcore_axis_name="s")
    out_type = tuple(
        jax.ShapeDtypeStruct((size, D), jnp.float32) for _ in range(n))
    scratch = (
        [pltpu.VMEM((chunk,), jnp.int32) for _ in range(n)]
        + [pltpu.VMEM((chunk, D), jnp.float32) for _ in range(n)]
        + [pltpu.SemaphoreType.DMA]
        + [pltpu.SemaphoreType.DMA for _ in range(n)]
    )

    def body(*refs):
        tbl = refs[:n]
        idx = refs[n:2 * n]
        out = refs[2 * n:3 * n]
        rest = refs[3 * n:]
        idx_v = rest[:n]
        rows_v = rest[n:2 * n]
        gsem = rest[2 * n]
        wsem = rest[2 * n + 1:]
        wid = lax.axis_index("s") * NC + lax.axis_index("c")
        base = wid * per_w

        def step(i, carry):
            off = pl.multiple_of(base + i * chunk, 8)
            for t in range(n):
                pltpu.sync_copy(idx[t].at[pl.ds(off, chunk)], idx_v[t])

                @pl.when(i > 0)
                def _(t=t):
                    pltpu.make_async_copy(
                        out[t].at[pl.ds(0, chunk)], rows_v[t], wsem[t]).wait()

                pltpu.async_copy(tbl[t].at[idx_v[t]], rows_v[t], gsem).wait()
                pltpu.async_copy(rows_v[t], out[t].at[pl.ds(off, chunk)],
                                 wsem[t])
            return carry

        lax.fori_loop(0, n_chunks, step, 0)
        for t in range(n):
            pltpu.make_async_copy(
                out[t].at[pl.ds(0, chunk)], rows_v[t], wsem[t]).wait()

    f = pl.kernel(body, out_type=out_type, mesh=mesh, scratch_types=scratch,
                  name=name)
    return f(*tables, *idxs)


def _sc_scatter(mf, mb, tails, heads, with_cnt, name):
    size = tails.shape[0]
    per_w = size // NW
    chunk = 200
    n_chunks = per_w // chunk
    mesh = plsc.VectorSubcoreMesh(core_axis_name="c", subcore_axis_name="s")
    out_type = [jax.ShapeDtypeStruct((NC, N, D), jnp.float32)]
    scratch = [
        pltpu.VMEM_SHARED((N, D), jnp.float32),
        pltpu.VMEM((chunk, D), jnp.float32),
        pltpu.VMEM((chunk,), jnp.int32),
    ]
    zeros_blk = jnp.zeros((_WFULL, D), jnp.float32)
    args = [mf, mb, tails, heads, zeros_blk]
    if with_cnt:
        out_type.append(jax.ShapeDtypeStruct((NC * N,), jnp.float32))
        scratch += [pltpu.VMEM_SHARED((N,), jnp.float32),
                    pltpu.VMEM((chunk,), jnp.float32),
                    pltpu.VMEM((N,), jnp.float32)]
        args += [jnp.zeros((N,), jnp.float32), jnp.ones((chunk,), jnp.float32)]

    def body(*refs):
        if with_cnt:
            (mf_h, mb_h, t_h, h_h, zb_h, zn_h, on_h, agg_o, cnt_o,
             agg_s, buf_v, idx_v, cnt_s, ones_v, cnt_v) = refs
        else:
            mf_h, mb_h, t_h, h_h, zb_h, agg_o, agg_s, buf_v, idx_v = refs
        c = lax.axis_index("c")
        s = lax.axis_index("s")
        wid = s * NC + c
        r0 = pl.multiple_of(s * _WFULL, 8)

        @pl.when(s < NS - 1)
        def _():
            pltpu.sync_copy(zb_h, agg_s.at[pl.ds(r0, _WFULL)])

        @pl.when(s == NS - 1)
        def _():
            pltpu.sync_copy(zb_h.at[pl.ds(0, _WLAST)],
                            agg_s.at[pl.ds(r0, _WLAST)])

        if with_cnt:
            @pl.when(s == 0)
            def _():
                pltpu.sync_copy(zn_h, cnt_v)
                pltpu.sync_copy(cnt_v, cnt_s)
            pltpu.sync_copy(on_h, ones_v)
        plsc.subcore_barrier()
        base = wid * per_w

        def step(i, carry):
            off = pl.multiple_of(base + i * chunk, 8)
            pltpu.sync_copy(t_h.at[pl.ds(off, chunk)], idx_v)
            pltpu.sync_copy(mf_h.at[pl.ds(off, chunk)], buf_v)
            pltpu.sync_copy(buf_v, agg_s.at[idx_v], add=True)
            if with_cnt:
                pltpu.sync_copy(ones_v, cnt_s.at[idx_v], add=True)
            pltpu.sync_copy(h_h.at[pl.ds(off, chunk)], idx_v)
            pltpu.sync_copy(mb_h.at[pl.ds(off, chunk)], buf_v)
            pltpu.sync_copy(buf_v, agg_s.at[idx_v], add=True)
            if with_cnt:
                pltpu.sync_copy(ones_v, cnt_s.at[idx_v], add=True)
            return carry

        lax.fori_loop(0, n_chunks, step, 0)
        plsc.subcore_barrier()

        @pl.when(s < NS - 1)
        def _():
            pltpu.sync_copy(agg_s.at[pl.ds(r0, _WFULL)],
                            agg_o.at[c, pl.ds(r0, _WFULL)])

        @pl.when(s == NS - 1)
        def _():
            pltpu.sync_copy(agg_s.at[pl.ds(r0, _WLAST)],
                            agg_o.at[c, pl.ds(r0, _WLAST)])

        if with_cnt:
            @pl.when(s == 0)
            def _():
                pltpu.sync_copy(cnt_s, cnt_v)
                pltpu.sync_copy(cnt_v,
                                cnt_o.at[pl.ds(pl.multiple_of(c * N, 8), N)])

    f = pl.kernel(body, out_type=tuple(out_type), mesh=mesh,
                  scratch_types=scratch, name=name)
    res = f(*args)
    return res if with_cnt else res[0]


def _layer_weights(W_eu, b_eu, W_mf, b_mf, W_mb, b_mb, ln_eu, l):
    Wh, We, Wt = W_eu[l, :D], W_eu[l, D:2 * D], W_eu[l, 2 * D:]
    Afh = W_mf[l, :D] + W_mf[l, 2 * D:3 * D]
    Afe = W_mf[l, D:2 * D] + W_mf[l, 2 * D:3 * D]
    Afp = W_mf[l, 3 * D:]
    Abt = W_mb[l, :D] + W_mb[l, 2 * D:3 * D]
    Abe = W_mb[l, D:2 * D] + W_mb[l, 2 * D:3 * D]
    Abp = W_mb[l, 3 * D:]
    Ws = jnp.stack([Wh, We, Wt, Afh, Afe, Afp, Abt, Abe, Abp])
    return Ws.astype(jnp.bfloat16), b_eu[l], b_mf[l], b_mb[l], ln_eu[l]


def kernel(entity_feat, ht, r_tensor, r_relative, W_ent, b_ent, ln_ent,
           rel_emb, dir_emb, W_eu, b_eu, ln_eu, W_mf, b_mf, W_mb, b_mb,
           ln_mp):
    heads = ht[:, 0]
    tails = ht[:, 1]
    T2 = (rel_emb[:, None, :] + dir_emb[None, :, :]).reshape(-1, D)
    idx_e = r_tensor * 2 + r_relative
    halves = tuple((i * MCH, (i + 1) * MCH) for i in range(NSPLIT))

    H = _encoder(entity_feat, W_ent, b_ent, ln_ent)
    gh = [_sc_gather([H, H, T2],
                     [heads[lo:hi], tails[lo:hi], idx_e[lo:hi]],
                     f"sc_gH0_{i}")
          for i, (lo, hi) in enumerate(halves)]
    Eh = [g[2] for g in gh]

    cnt3s = None
    for l in range(L):
        Ws, beu, bmf, bmb, lnr = _layer_weights(
            W_eu, b_eu, W_mf, b_mf, W_mb, b_mb, ln_eu, l)
        first = l == 0
        eouts = [_edge(gh[i][0], gh[i][1], Eh[i], Ws, beu, bmf, bmb, lnr,
                       out_e=first, name=f"tc_edge{l}_{i}")
                 for i in range(NSPLIT)]
        if first:
            Eh = [eo[0] for eo in eouts]
            scs = [_sc_scatter(eo[1], eo[2], tails[lo:hi], heads[lo:hi],
                               True, f"sc_scat{l}_{i}")
                   for i, (eo, (lo, hi)) in enumerate(zip(eouts, halves))]
            aggps = [s[0] for s in scs]
            cnt3s = [s[1].reshape(NC, N, 1) for s in scs]
        else:
            scs = [_sc_scatter(eo[0], eo[1], tails[lo:hi], heads[lo:hi],
                               False, f"sc_scat{l}_{i}")
                   for i, (eo, (lo, hi)) in enumerate(zip(eouts, halves))]
            aggps = [s for s in scs]
        H = _node(aggps, cnt3s, H, ln_mp[l], f"tc_node{l}")
        if first:
            gh = [_sc_gather([H, H], [heads[lo:hi], tails[lo:hi]],
                             f"sc_gH1_{i}")
                  for i, (lo, hi) in enumerate(halves)]
    return H


# speedup vs baseline: 1.5331x; 1.0307x over previous
import functools

import jax
import jax.numpy as jnp
from jax import lax
from jax.experimental import pallas as pl
from jax.experimental.pallas import tpu as pltpu
from jax.experimental.pallas import tpu_sc as plsc

N = 10000
M = 320000
D_IN = 768
D = 128
L = 2

NC = 2
NS = 16
NW = NC * NS
ROWS_PER_TILE = N // NS
_WFULL = 632
_WLAST = N - (NS - 1) * _WFULL
NSPLIT = 5
MCH = M // NSPLIT


def _lrelu(x):
    return jnp.where(x >= 0, x, 0.01 * x)


def _ln(x, g, b):
    mu = jnp.mean(x, axis=-1, keepdims=True)
    var = jnp.mean((x - mu) * (x - mu), axis=-1, keepdims=True)
    return (x - mu) * lax.rsqrt(var + 1e-5) * g + b


def _chunk_for(per_w):
    for c in (400, 200, 100, 40):
        if per_w % c == 0 and c % 8 == 0:
            return c
    raise ValueError(per_w)


def _encoder(x, W, b, ln):
    BN = 1000

    def body(x_ref, w_ref, b_ref, ln_ref, o_ref):
        y = jnp.dot(x_ref[...], w_ref[...], preferred_element_type=jnp.float32)
        y = _lrelu(y + b_ref[...])
        o_ref[...] = _ln(y, ln_ref[0:1], ln_ref[1:2])

    return pl.pallas_call(
        body,
        grid=(N // BN,),
        in_specs=[
            pl.BlockSpec((BN, D_IN), lambda i: (i, 0)),
            pl.BlockSpec((D_IN, D), lambda i: (0, 0)),
            pl.BlockSpec((1, D), lambda i: (0, 0)),
            pl.BlockSpec((2, D), lambda i: (0, 0)),
        ],
        out_specs=pl.BlockSpec((BN, D), lambda i: (i, 0)),
        out_shape=jax.ShapeDtypeStruct((N, D), jnp.float32),
        name="tc_encoder",
    )(x, W, b.reshape(1, D), ln)


def _edge(Hh, Ht, Eg, Ws, beu, bmf, bmb, lnr, out_e, name):
    size = Hh.shape[0]
    BE = 640

    def body(hh, ht, eg, ws, b1, b2, b3, lnref, *outs):
        Hhf = hh[...]
        Htf = ht[...]
        Ef = eg[...].astype(jnp.float32)
        Hh_ = Hhf.astype(jnp.bfloat16)
        Ht_ = Htf.astype(jnp.bfloat16)
        Eb = Ef.astype(jnp.bfloat16)
        dot = functools.partial(jnp.dot, preferred_element_type=jnp.float32)
        T = dot(Hh_, ws[0]) + dot(Eb, ws[1]) + dot(Ht_, ws[2]) + b1[...]
        Enf = _ln(_lrelu(T) + Ef, lnref[0:1], lnref[1:2])
        Enb = Enf.astype(jnp.bfloat16)
        Pf = (Hhf * Enf).astype(jnp.bfloat16)
        Pb = (Htf * Enf).astype(jnp.bfloat16)
        mf = dot(Hh_, ws[3]) + dot(Enb, ws[4]) + dot(Pf, ws[5]) + b2[...]
        mb = dot(Ht_, ws[6]) + dot(Enb, ws[7]) + dot(Pb, ws[8]) + b3[...]
        if out_e:
            outs[0][...] = Enb
            outs[1][...] = mf
            outs[2][...] = mb
        else:
            outs[0][...] = mf
            outs[1][...] = mb

    eb = pl.BlockSpec((BE, D), lambda i: (i, 0))

    def cb(shape):
        return pl.BlockSpec(shape, lambda i, _s=shape: tuple(0 for _ in _s))

    in_specs = [eb, eb, eb,
                cb((9, D, D)), cb((1, D)), cb((1, D)), cb((1, D)), cb((2, D))]
    args = [Hh, Ht, Eg, Ws,
            beu.reshape(1, D), bmf.reshape(1, D), bmb.reshape(1, D), lnr]
    out_shape = [jax.ShapeDtypeStruct((size, D), jnp.float32),
                 jax.ShapeDtypeStruct((size, D), jnp.float32)]
    if out_e:
        out_shape = [jax.ShapeDtypeStruct((size, D), jnp.bfloat16)] + out_shape
    return pl.pallas_call(
        body,
        grid=(size // BE,),
        in_specs=in_specs,
        out_specs=[eb] * len(out_shape),
        out_shape=out_shape,
        name=name,
    )(*args)


def _node(aggps, cnt3s, H, lnr, name):
    BN = 1000
    na, nc = len(aggps), len(cnt3s)

    def body(*refs):
        aggs = refs[:na]
        cnts = refs[na:na + nc]
        h_ref, ln_ref, o_ref = refs[na + nc:]
        a = aggs[0][0] + aggs[0][1]
        for r in aggs[1:]:
            a = a + r[0] + r[1]
        cnt = cnts[0][0] + cnts[0][1]
        for r in cnts[1:]:
            cnt = cnt + r[0] + r[1]
        a = a / jnp.maximum(cnt, 1.0)
        x = _lrelu(a) + h_ref[...]
        o_ref[...] = _ln(x, ln_ref[0:1], ln_ref[1:2])

    return pl.pallas_call(
        body,
        grid=(N // BN,),
        in_specs=(
            [pl.BlockSpec((2, BN, D), lambda i: (0, i, 0))] * na
            + [pl.BlockSpec((2, BN, 1), lambda i: (0, i, 0))] * nc
            + [pl.BlockSpec((BN, D), lambda i: (i, 0)),
               pl.BlockSpec((2, D), lambda i: (0, 0))]
        ),
        out_specs=pl.BlockSpec((BN, D), lambda i: (i, 0)),
        out_shape=jax.ShapeDtypeStruct((N, D), jnp.float32),
        name=name,
    )(*aggps, *cnt3s, H, lnr)


def _sc_gather(tables, idxs, name):
    n = len(tables)
    size = idxs[0].shape[0]
    per_w = size // NW
    chunk = 200 if n >= 3 else _chunk_for(per_w)
    n_chunks = per_w // chunk
    mesh = plsc.VectorSubcoreMesh(core_axis_name="c", subcore_axis_name="s")
    out_type = tuple(
        jax.ShapeDtypeStruct((size, D), jnp.float32) for _ in range(n))
    scratch = (
        [pltpu.VMEM((chunk,), jnp.int32) for _ in range(n)]
        + [pltpu.VMEM((chunk, D), jnp.float32) for _ in range(n)]
        + [pltpu.SemaphoreType.DMA]
        + [pltpu.SemaphoreType.DMA for _ in range(n)]
    )

    def body(*refs):
        tbl = refs[:n]
        idx = refs[n:2 * n]
        out = refs[2 * n:3 * n]
        rest = refs[3 * n:]
        idx_v = rest[:n]
        rows_v = rest[n:2 * n]
        gsem = rest[2 * n]
        wsem = rest[2 * n + 1:]
        wid = lax.axis_index("s") * NC + lax.axis_index("c")
        base = wid * per_w

        def step(i, carry):
            off = pl.multiple_of(base + i * chunk, 8)
            for t in range(n):
                pltpu.sync_copy(idx[t].at[pl.ds(off, chunk)], idx_v[t])

                @pl.when(i > 0)
                def _(t=t):
                    pltpu.make_async_copy(
                        out[t].at[pl.ds(0, chunk)], rows_v[t], wsem[t]).wait()

                pltpu.async_copy(tbl[t].at[idx_v[t]], rows_v[t], gsem).wait()
                pltpu.async_copy(rows_v[t], out[t].at[pl.ds(off, chunk)],
                                 wsem[t])
            return carry

        lax.fori_loop(0, n_chunks, step, 0)
        for t in range(n):
            pltpu.make_async_copy(
                out[t].at[pl.ds(0, chunk)], rows_v[t], wsem[t]).wait()

    f = pl.kernel(body, out_type=out_type, mesh=mesh, scratch_types=scratch,
                  name=name)
    return f(*tables, *idxs)


def _sc_scatter(mf, mb, tails, heads, with_cnt, name):
    size = tails.shape[0]
    per_w = size // NW
    chunk = 80
    n_chunks = per_w // chunk
    mesh = plsc.VectorSubcoreMesh(core_axis_name="c", subcore_axis_name="s")
    out_type = [jax.ShapeDtypeStruct((NC, N, D), jnp.float32)]
    scratch = [
        pltpu.VMEM_SHARED((N, D), jnp.float32),
        pltpu.VMEM((chunk, D), jnp.float32),
        pltpu.VMEM((chunk, D), jnp.float32),
        pltpu.VMEM((chunk,), jnp.int32),
        pltpu.VMEM((chunk,), jnp.int32),
    ] + [pltpu.SemaphoreType.DMA] * 6
    zeros_blk = jnp.zeros((_WFULL, D), jnp.float32)
    args = [mf, mb, tails, heads, zeros_blk]
    if with_cnt:
        out_type.append(jax.ShapeDtypeStruct((NC * N,), jnp.float32))
        scratch += [pltpu.VMEM_SHARED((N,), jnp.float32),
                    pltpu.VMEM((chunk,), jnp.float32),
                    pltpu.VMEM((N,), jnp.float32),
                    pltpu.SemaphoreType.DMA, pltpu.SemaphoreType.DMA]
        args += [jnp.zeros((N,), jnp.float32), jnp.ones((chunk,), jnp.float32)]

    def body(*refs):
        if with_cnt:
            (mf_h, mb_h, t_h, h_h, zb_h, zn_h, on_h, agg_o, cnt_o,
             agg_s, b0, b1, i0, i1, lsi0, lsb0, lsi1, lsb1, ss0, ss1,
             cnt_s, ones_v, cnt_v, cs0, cs1) = refs
        else:
            (mf_h, mb_h, t_h, h_h, zb_h, agg_o,
             agg_s, b0, b1, i0, i1, lsi0, lsb0, lsi1, lsb1, ss0, ss1) = refs
        c = lax.axis_index("c")
        s = lax.axis_index("s")
        wid = s * NC + c
        r0 = pl.multiple_of(s * _WFULL, 8)

        @pl.when(s < NS - 1)
        def _():
            pltpu.sync_copy(zb_h, agg_s.at[pl.ds(r0, _WFULL)])

        @pl.when(s == NS - 1)
        def _():
            pltpu.sync_copy(zb_h.at[pl.ds(0, _WLAST)],
                            agg_s.at[pl.ds(r0, _WLAST)])

        if with_cnt:
            @pl.when(s == 0)
            def _():
                pltpu.sync_copy(zn_h, cnt_v)
                pltpu.sync_copy(cnt_v, cnt_s)
            pltpu.sync_copy(on_h, ones_v)
        plsc.subcore_barrier()
        base = wid * per_w

        pltpu.async_copy(t_h.at[pl.ds(base, chunk)], i0, lsi0)
        pltpu.async_copy(mf_h.at[pl.ds(base, chunk)], b0, lsb0)
        pltpu.async_copy(h_h.at[pl.ds(base, chunk)], i1, lsi1)
        pltpu.async_copy(mb_h.at[pl.ds(base, chunk)], b1, lsb1)

        def step(j, carry):
            off_n = pl.multiple_of(base + (j + 1) * chunk, 8)
            pltpu.make_async_copy(t_h.at[pl.ds(0, chunk)], i0, lsi0).wait()
            pltpu.make_async_copy(mf_h.at[pl.ds(0, chunk)], b0, lsb0).wait()
            d0 = pltpu.async_copy(b0, agg_s.at[i0], ss0, add=True)
            if with_cnt:
                dc0 = pltpu.async_copy(ones_v, cnt_s.at[i0], cs0, add=True)
            pltpu.make_async_copy(h_h.at[pl.ds(0, chunk)], i1, lsi1).wait()
            pltpu.make_async_copy(mb_h.at[pl.ds(0, chunk)], b1, lsb1).wait()
            d1 = pltpu.async_copy(b1, agg_s.at[i1], ss1, add=True)
            if with_cnt:
                dc1 = pltpu.async_copy(ones_v, cnt_s.at[i1], cs1, add=True)
            d0.wait()
            if with_cnt:
                dc0.wait()

            @pl.when(j < n_chunks - 1)
            def _():
                pltpu.async_copy(t_h.at[pl.ds(off_n, chunk)], i0, lsi0)
                pltpu.async_copy(mf_h.at[pl.ds(off_n, chunk)], b0, lsb0)

            d1.wait()
            if with_cnt:
                dc1.wait()

            @pl.when(j < n_chunks - 1)
            def _():
                pltpu.async_copy(h_h.at[pl.ds(off_n, chunk)], i1, lsi1)
                pltpu.async_copy(mb_h.at[pl.ds(off_n, chunk)], b1, lsb1)
            return carry

        lax.fori_loop(0, n_chunks, step, 0)
        plsc.subcore_barrier()

        @pl.when(s < NS - 1)
        def _():
            pltpu.sync_copy(agg_s.at[pl.ds(r0, _WFULL)],
                            agg_o.at[c, pl.ds(r0, _WFULL)])

        @pl.when(s == NS - 1)
        def _():
            pltpu.sync_copy(agg_s.at[pl.ds(r0, _WLAST)],
                            agg_o.at[c, pl.ds(r0, _WLAST)])

        if with_cnt:
            @pl.when(s == 0)
            def _():
                pltpu.sync_copy(cnt_s, cnt_v)
                pltpu.sync_copy(cnt_v,
                                cnt_o.at[pl.ds(pl.multiple_of(c * N, 8), N)])

    f = pl.kernel(body, out_type=tuple(out_type), mesh=mesh,
                  scratch_types=scratch, name=name)
    res = f(*args)
    return res if with_cnt else res[0]


def _layer_weights(W_eu, b_eu, W_mf, b_mf, W_mb, b_mb, ln_eu, l):
    Wh, We, Wt = W_eu[l, :D], W_eu[l, D:2 * D], W_eu[l, 2 * D:]
    Afh = W_mf[l, :D] + W_mf[l, 2 * D:3 * D]
    Afe = W_mf[l, D:2 * D] + W_mf[l, 2 * D:3 * D]
    Afp = W_mf[l, 3 * D:]
    Abt = W_mb[l, :D] + W_mb[l, 2 * D:3 * D]
    Abe = W_mb[l, D:2 * D] + W_mb[l, 2 * D:3 * D]
    Abp = W_mb[l, 3 * D:]
    Ws = jnp.stack([Wh, We, Wt, Afh, Afe, Afp, Abt, Abe, Abp])
    return Ws.astype(jnp.bfloat16), b_eu[l], b_mf[l], b_mb[l], ln_eu[l]


def kernel(entity_feat, ht, r_tensor, r_relative, W_ent, b_ent, ln_ent,
           rel_emb, dir_emb, W_eu, b_eu, ln_eu, W_mf, b_mf, W_mb, b_mb,
           ln_mp):
    heads = ht[:, 0]
    tails = ht[:, 1]
    T2 = (rel_emb[:, None, :] + dir_emb[None, :, :]).reshape(-1, D)
    idx_e = r_tensor * 2 + r_relative
    halves = tuple((i * MCH, (i + 1) * MCH) for i in range(NSPLIT))

    H = _encoder(entity_feat, W_ent, b_ent, ln_ent)
    gh = [_sc_gather([H, H, T2],
                     [heads[lo:hi], tails[lo:hi], idx_e[lo:hi]],
                     f"sc_gH0_{i}")
          for i, (lo, hi) in enumerate(halves)]
    Eh = [g[2] for g in gh]

    cnt3s = None
    for l in range(L):
        Ws, beu, bmf, bmb, lnr = _layer_weights(
            W_eu, b_eu, W_mf, b_mf, W_mb, b_mb, ln_eu, l)
        first = l == 0
        eouts = [_edge(gh[i][0], gh[i][1], Eh[i], Ws, beu, bmf, bmb, lnr,
                       out_e=first, name=f"tc_edge{l}_{i}")
                 for i in range(NSPLIT)]
        if first:
            Eh = [eo[0] for eo in eouts]
            scs = [_sc_scatter(eo[1], eo[2], tails[lo:hi], heads[lo:hi],
                               True, f"sc_scat{l}_{i}")
                   for i, (eo, (lo, hi)) in enumerate(zip(eouts, halves))]
            aggps = [s[0] for s in scs]
            cnt3s = [s[1].reshape(NC, N, 1) for s in scs]
        else:
            scs = [_sc_scatter(eo[0], eo[1], tails[lo:hi], heads[lo:hi],
                               False, f"sc_scat{l}_{i}")
                   for i, (eo, (lo, hi)) in enumerate(zip(eouts, halves))]
            aggps = [s for s in scs]
        H = _node(aggps, cnt3s, H, ln_mp[l], f"tc_node{l}")
        if first:
            gh = [_sc_gather([H, H], [heads[lo:hi], tails[lo:hi]],
                             f"sc_gH1_{i}")
                  for i, (lo, hi) in enumerate(halves)]
    return H


# speedup vs baseline: 1.6623x; 1.0843x over previous
import functools

import jax
import jax.numpy as jnp
from jax import lax
from jax.experimental import pallas as pl
from jax.experimental.pallas import tpu as pltpu
from jax.experimental.pallas import tpu_sc as plsc

N = 10000
M = 320000
D_IN = 768
D = 128
L = 2

NC = 2
NS = 16
NW = NC * NS
ROWS_PER_TILE = N // NS
_WFULL = 632
_WLAST = N - (NS - 1) * _WFULL
NSPLIT = 5
MCH = M // NSPLIT


def _lrelu(x):
    return jnp.where(x >= 0, x, 0.01 * x)


def _ln(x, g, b):
    mu = jnp.mean(x, axis=-1, keepdims=True)
    var = jnp.mean((x - mu) * (x - mu), axis=-1, keepdims=True)
    return (x - mu) * lax.rsqrt(var + 1e-5) * g + b


def _chunk_for(per_w):
    for c in (400, 200, 100, 40):
        if per_w % c == 0 and c % 8 == 0:
            return c
    raise ValueError(per_w)


def _encoder(x, W, b, ln):
    BN = 1000

    def body(x_ref, w_ref, b_ref, ln_ref, o_ref):
        y = jnp.dot(x_ref[...], w_ref[...], preferred_element_type=jnp.float32)
        y = _lrelu(y + b_ref[...])
        o_ref[...] = _ln(y, ln_ref[0:1], ln_ref[1:2])

    return pl.pallas_call(
        body,
        grid=(N // BN,),
        in_specs=[
            pl.BlockSpec((BN, D_IN), lambda i: (i, 0)),
            pl.BlockSpec((D_IN, D), lambda i: (0, 0)),
            pl.BlockSpec((1, D), lambda i: (0, 0)),
            pl.BlockSpec((2, D), lambda i: (0, 0)),
        ],
        out_specs=pl.BlockSpec((BN, D), lambda i: (i, 0)),
        out_shape=jax.ShapeDtypeStruct((N, D), jnp.float32),
        name="tc_encoder",
    )(x, W, b.reshape(1, D), ln)


def _edge(Hh, Ht, Eg, Ws, beu, bmf, bmb, lnr, out_e, name):
    size = Hh.shape[0]
    BE = 640

    def body(hh, ht, eg, ws, b1, b2, b3, lnref, *outs):
        Hhf = hh[...]
        Htf = ht[...]
        Ef = eg[...].astype(jnp.float32)
        Hh_ = Hhf.astype(jnp.bfloat16)
        Ht_ = Htf.astype(jnp.bfloat16)
        Eb = Ef.astype(jnp.bfloat16)
        dot = functools.partial(jnp.dot, preferred_element_type=jnp.float32)
        T = dot(Hh_, ws[0]) + dot(Eb, ws[1]) + dot(Ht_, ws[2]) + b1[...]
        Enf = _ln(_lrelu(T) + Ef, lnref[0:1], lnref[1:2])
        Enb = Enf.astype(jnp.bfloat16)
        Pf = (Hhf * Enf).astype(jnp.bfloat16)
        Pb = (Htf * Enf).astype(jnp.bfloat16)
        mf = dot(Hh_, ws[3]) + dot(Enb, ws[4]) + dot(Pf, ws[5]) + b2[...]
        mb = dot(Ht_, ws[6]) + dot(Enb, ws[7]) + dot(Pb, ws[8]) + b3[...]
        if out_e:
            outs[0][...] = Enb
            outs[1][...] = mf
            outs[2][...] = mb
        else:
            outs[0][...] = mf
            outs[1][...] = mb

    eb = pl.BlockSpec((BE, D), lambda i: (i, 0))

    def cb(shape):
        return pl.BlockSpec(shape, lambda i, _s=shape: tuple(0 for _ in _s))

    in_specs = [eb, eb, eb,
                cb((9, D, D)), cb((1, D)), cb((1, D)), cb((1, D)), cb((2, D))]
    args = [Hh, Ht, Eg, Ws,
            beu.reshape(1, D), bmf.reshape(1, D), bmb.reshape(1, D), lnr]
    out_shape = [jax.ShapeDtypeStruct((size, D), jnp.float32),
                 jax.ShapeDtypeStruct((size, D), jnp.float32)]
    if out_e:
        out_shape = [jax.ShapeDtypeStruct((size, D), jnp.bfloat16)] + out_shape
    return pl.pallas_call(
        body,
        grid=(size // BE,),
        in_specs=in_specs,
        out_specs=[eb] * len(out_shape),
        out_shape=out_shape,
        name=name,
    )(*args)


def _node(aggps, cnt3, H, lnr, name):
    BN = 1000
    na = len(aggps)

    def body(*refs):
        aggs = refs[:na]
        c_ref, h_ref, ln_ref, o_ref = refs[na:]
        a = aggs[0][0] + aggs[0][1]
        for r in aggs[1:]:
            a = a + r[0] + r[1]
        a = a / jnp.maximum(c_ref[...], 1.0)
        x = _lrelu(a) + h_ref[...]
        o_ref[...] = _ln(x, ln_ref[0:1], ln_ref[1:2])

    return pl.pallas_call(
        body,
        grid=(N // BN,),
        in_specs=(
            [pl.BlockSpec((2, BN, D), lambda i: (0, i, 0))] * na
            + [pl.BlockSpec((BN, 1), lambda i: (i, 0)),
               pl.BlockSpec((BN, D), lambda i: (i, 0)),
               pl.BlockSpec((2, D), lambda i: (0, 0))]
        ),
        out_specs=pl.BlockSpec((BN, D), lambda i: (i, 0)),
        out_shape=jax.ShapeDtypeStruct((N, D), jnp.float32),
        name=name,
    )(*aggps, cnt3, H, lnr)


def _sc_gather(tables, idxs, name):
    n = len(tables)
    size = idxs[0].shape[0]
    per_w = size // NW
    chunk = 80
    n_chunks = per_w // chunk
    uniq, tmap = [], []
    for t in tables:
        for u, ut in enumerate(uniq):
            if ut is t:
                tmap.append(u)
                break
        else:
            tmap.append(len(uniq))
            uniq.append(t)
    nu = len(uniq)
    rows_n = [t.shape[0] for t in uniq]
    splits = []
    for rt in rows_n:
        full = -(-rt // NS)
        full += (-full) % 8
        splits.append((full, rt - (NS - 1) * full))
    mesh = plsc.VectorSubcoreMesh(core_axis_name="c", subcore_axis_name="s")
    out_type = tuple(
        jax.ShapeDtypeStruct((size, D), jnp.float32) for _ in range(n))
    scratch = (
        [pltpu.VMEM_SHARED((rt, D), jnp.float32) for rt in rows_n]
        + [pltpu.VMEM((chunk,), jnp.int32) for _ in range(n)]
        + [pltpu.VMEM((chunk, D), jnp.float32) for _ in range(n)]
        + [pltpu.SemaphoreType.DMA]
        + [pltpu.SemaphoreType.DMA for _ in range(n)]
    )

    def body(*refs):
        tbl = refs[:nu]
        idx = refs[nu:nu + n]
        out = refs[nu + n:nu + 2 * n]
        rest = refs[nu + 2 * n:]
        stage = rest[:nu]
        idx_v = rest[nu:nu + n]
        rows_v = rest[nu + n:nu + 2 * n]
        gsem = rest[nu + 2 * n]
        wsem = rest[nu + 2 * n + 1:]
        s = lax.axis_index("s")
        wid = s * NC + lax.axis_index("c")
        base = wid * per_w

        for u in range(nu):
            full, last = splits[u]
            rt0 = pl.multiple_of(s * full, 8)

            @pl.when(s < NS - 1)
            def _(u=u, full=full, rt0=rt0):
                pltpu.sync_copy(tbl[u].at[pl.ds(rt0, full)],
                                stage[u].at[pl.ds(rt0, full)])

            @pl.when(s == NS - 1)
            def _(u=u, last=last, rt0=rt0):
                pltpu.sync_copy(tbl[u].at[pl.ds(rt0, last)],
                                stage[u].at[pl.ds(rt0, last)])
        plsc.subcore_barrier()

        def step(i, carry):
            off = pl.multiple_of(base + i * chunk, 8)
            for t in range(n):
                pltpu.sync_copy(idx[t].at[pl.ds(off, chunk)], idx_v[t])

                @pl.when(i > 0)
                def _(t=t):
                    pltpu.make_async_copy(
                        out[t].at[pl.ds(0, chunk)], rows_v[t], wsem[t]).wait()

                pltpu.async_copy(stage[tmap[t]].at[idx_v[t]], rows_v[t],
                                 gsem).wait()
                pltpu.async_copy(rows_v[t], out[t].at[pl.ds(off, chunk)],
                                 wsem[t])
            return carry

        lax.fori_loop(0, n_chunks, step, 0)
        for t in range(n):
            pltpu.make_async_copy(
                out[t].at[pl.ds(0, chunk)], rows_v[t], wsem[t]).wait()

    f = pl.kernel(body, out_type=out_type, mesh=mesh, scratch_types=scratch,
                  name=name)
    return f(*uniq, *idxs)


def _sc_scatter(mf, mb, tails, heads, with_cnt, name):
    size = tails.shape[0]
    per_w = size // NW
    chunk = 80
    n_chunks = per_w // chunk
    mesh = plsc.VectorSubcoreMesh(core_axis_name="c", subcore_axis_name="s")
    out_type = [jax.ShapeDtypeStruct((NC, N, D), jnp.float32)]
    scratch = [
        pltpu.VMEM_SHARED((N, D), jnp.float32),
        pltpu.VMEM((chunk, D), jnp.float32),
        pltpu.VMEM((chunk, D), jnp.float32),
        pltpu.VMEM((chunk,), jnp.int32),
        pltpu.VMEM((chunk,), jnp.int32),
    ] + [pltpu.SemaphoreType.DMA] * 6
    zeros_blk = jnp.zeros((_WFULL, D), jnp.float32)
    args = [mf, mb, tails, heads, zeros_blk]
    if with_cnt:
        out_type.append(jax.ShapeDtypeStruct((NC * N,), jnp.float32))
        scratch += [pltpu.VMEM_SHARED((N,), jnp.float32),
                    pltpu.VMEM((chunk,), jnp.float32),
                    pltpu.VMEM((N,), jnp.float32),
                    pltpu.SemaphoreType.DMA, pltpu.SemaphoreType.DMA]
        args += [jnp.zeros((N,), jnp.float32), jnp.ones((chunk,), jnp.float32)]

    def body(*refs):
        if with_cnt:
            (mf_h, mb_h, t_h, h_h, zb_h, zn_h, on_h, agg_o, cnt_o,
             agg_s, b0, b1, i0, i1, lsi0, lsb0, lsi1, lsb1, ss0, ss1,
             cnt_s, ones_v, cnt_v, cs0, cs1) = refs
        else:
            (mf_h, mb_h, t_h, h_h, zb_h, agg_o,
             agg_s, b0, b1, i0, i1, lsi0, lsb0, lsi1, lsb1, ss0, ss1) = refs
        c = lax.axis_index("c")
        s = lax.axis_index("s")
        wid = s * NC + c
        r0 = pl.multiple_of(s * _WFULL, 8)

        @pl.when(s < NS - 1)
        def _():
            pltpu.sync_copy(zb_h, agg_s.at[pl.ds(r0, _WFULL)])

        @pl.when(s == NS - 1)
        def _():
            pltpu.sync_copy(zb_h.at[pl.ds(0, _WLAST)],
                            agg_s.at[pl.ds(r0, _WLAST)])

        if with_cnt:
            @pl.when(s == 0)
            def _():
                pltpu.sync_copy(zn_h, cnt_v)
                pltpu.sync_copy(cnt_v, cnt_s)
            pltpu.sync_copy(on_h, ones_v)
        plsc.subcore_barrier()
        base = wid * per_w

        pltpu.async_copy(t_h.at[pl.ds(base, chunk)], i0, lsi0)
        pltpu.async_copy(mf_h.at[pl.ds(base, chunk)], b0, lsb0)
        pltpu.async_copy(h_h.at[pl.ds(base, chunk)], i1, lsi1)
        pltpu.async_copy(mb_h.at[pl.ds(base, chunk)], b1, lsb1)

        def step(j, carry):
            off_n = pl.multiple_of(base + (j + 1) * chunk, 8)
            pltpu.make_async_copy(t_h.at[pl.ds(0, chunk)], i0, lsi0).wait()
            pltpu.make_async_copy(mf_h.at[pl.ds(0, chunk)], b0, lsb0).wait()
            d0 = pltpu.async_copy(b0, agg_s.at[i0], ss0, add=True)
            if with_cnt:
                dc0 = pltpu.async_copy(ones_v, cnt_s.at[i0], cs0, add=True)
            pltpu.make_async_copy(h_h.at[pl.ds(0, chunk)], i1, lsi1).wait()
            pltpu.make_async_copy(mb_h.at[pl.ds(0, chunk)], b1, lsb1).wait()
            d1 = pltpu.async_copy(b1, agg_s.at[i1], ss1, add=True)
            if with_cnt:
                dc1 = pltpu.async_copy(ones_v, cnt_s.at[i1], cs1, add=True)
            d0.wait()
            if with_cnt:
                dc0.wait()

            @pl.when(j < n_chunks - 1)
            def _():
                pltpu.async_copy(t_h.at[pl.ds(off_n, chunk)], i0, lsi0)
                pltpu.async_copy(mf_h.at[pl.ds(off_n, chunk)], b0, lsb0)

            d1.wait()
            if with_cnt:
                dc1.wait()

            @pl.when(j < n_chunks - 1)
            def _():
                pltpu.async_copy(h_h.at[pl.ds(off_n, chunk)], i1, lsi1)
                pltpu.async_copy(mb_h.at[pl.ds(off_n, chunk)], b1, lsb1)
            return carry

        lax.fori_loop(0, n_chunks, step, 0)
        plsc.subcore_barrier()

        @pl.when(s < NS - 1)
        def _():
            pltpu.sync_copy(agg_s.at[pl.ds(r0, _WFULL)],
                            agg_o.at[c, pl.ds(r0, _WFULL)])

        @pl.when(s == NS - 1)
        def _():
            pltpu.sync_copy(agg_s.at[pl.ds(r0, _WLAST)],
                            agg_o.at[c, pl.ds(r0, _WLAST)])

        if with_cnt:
            @pl.when(s == 0)
            def _():
                pltpu.sync_copy(cnt_s, cnt_v)
                pltpu.sync_copy(cnt_v,
                                cnt_o.at[pl.ds(pl.multiple_of(c * N, 8), N)])

    f = pl.kernel(body, out_type=tuple(out_type), mesh=mesh,
                  scratch_types=scratch, name=name)
    res = f(*args)
    return res if with_cnt else res[0]


def _layer_weights(W_eu, b_eu, W_mf, b_mf, W_mb, b_mb, ln_eu, l):
    Wh, We, Wt = W_eu[l, :D], W_eu[l, D:2 * D], W_eu[l, 2 * D:]
    Afh = W_mf[l, :D] + W_mf[l, 2 * D:3 * D]
    Afe = W_mf[l, D:2 * D] + W_mf[l, 2 * D:3 * D]
    Afp = W_mf[l, 3 * D:]
    Abt = W_mb[l, :D] + W_mb[l, 2 * D:3 * D]
    Abe = W_mb[l, D:2 * D] + W_mb[l, 2 * D:3 * D]
    Abp = W_mb[l, 3 * D:]
    Ws = jnp.stack([Wh, We, Wt, Afh, Afe, Afp, Abt, Abe, Abp])
    return Ws.astype(jnp.bfloat16), b_eu[l], b_mf[l], b_mb[l], ln_eu[l]


def kernel(entity_feat, ht, r_tensor, r_relative, W_ent, b_ent, ln_ent,
           rel_emb, dir_emb, W_eu, b_eu, ln_eu, W_mf, b_mf, W_mb, b_mb,
           ln_mp):
    heads = ht[:, 0]
    tails = ht[:, 1]
    T2 = (rel_emb[:, None, :] + dir_emb[None, :, :]).reshape(-1, D)
    idx_e = r_tensor * 2 + r_relative
    halves = tuple((i * MCH, (i + 1) * MCH) for i in range(NSPLIT))

    H = _encoder(entity_feat, W_ent, b_ent, ln_ent)
    gh = [_sc_gather([H, H, T2],
                     [heads[lo:hi], tails[lo:hi], idx_e[lo:hi]],
                     f"sc_gH0_{i}")
          for i, (lo, hi) in enumerate(halves)]
    Eh = [g[2] for g in gh]

    cnt3 = None
    for l in range(L):
        Ws, beu, bmf, bmb, lnr = _layer_weights(
            W_eu, b_eu, W_mf, b_mf, W_mb, b_mb, ln_eu, l)
        first = l == 0
        eouts = [_edge(gh[i][0], gh[i][1], Eh[i], Ws, beu, bmf, bmb, lnr,
                       out_e=first, name=f"tc_edge{l}_{i}")
                 for i in range(NSPLIT)]
        if first:
            Eh = [eo[0] for eo in eouts]
            scs = [_sc_scatter(eo[1], eo[2], tails[lo:hi], heads[lo:hi],
                               True, f"sc_scat{l}_{i}")
                   for i, (eo, (lo, hi)) in enumerate(zip(eouts, halves))]
            aggps = [s[0] for s in scs]
            cnt_flat = scs[0][1]
            for s_ in scs[1:]:
                cnt_flat = cnt_flat + s_[1]
            cnt3 = (cnt_flat[:N] + cnt_flat[N:]).reshape(N, 1)
        else:
            scs = [_sc_scatter(eo[0], eo[1], tails[lo:hi], heads[lo:hi],
                               False, f"sc_scat{l}_{i}")
                   for i, (eo, (lo, hi)) in enumerate(zip(eouts, halves))]
            aggps = [s for s in scs]
        H = _node(aggps, cnt3, H, ln_mp[l], f"tc_node{l}")
        if first:
            gh = [_sc_gather([H, H], [heads[lo:hi], tails[lo:hi]],
                             f"sc_gH1_{i}")
                  for i, (lo, hi) in enumerate(halves)]
    return H


# speedup vs baseline: 1.7029x; 1.0244x over previous
import functools

import jax
import jax.numpy as jnp
from jax import lax
from jax.experimental import pallas as pl
from jax.experimental.pallas import tpu as pltpu
from jax.experimental.pallas import tpu_sc as plsc

N = 10000
M = 320000
D_IN = 768
D = 128
L = 2

NC = 2
NS = 16
NW = NC * NS
ROWS_PER_TILE = N // NS
_WFULL = 632
_WLAST = N - (NS - 1) * _WFULL
_SIZES = (38400, 76800, 76800, 76800, 51200)
NSPLIT = len(_SIZES)


def _lrelu(x):
    return jnp.where(x >= 0, x, 0.01 * x)


def _ln(x, g, b):
    mu = jnp.mean(x, axis=-1, keepdims=True)
    var = jnp.mean((x - mu) * (x - mu), axis=-1, keepdims=True)
    return (x - mu) * lax.rsqrt(var + 1e-5) * g + b


def _chunk_for(per_w):
    for c in (400, 200, 100, 40):
        if per_w % c == 0 and c % 8 == 0:
            return c
    raise ValueError(per_w)


def _encoder(x, W, b, ln):
    BN = 1000

    def body(x_ref, w_ref, b_ref, ln_ref, o_ref):
        y = jnp.dot(x_ref[...], w_ref[...], preferred_element_type=jnp.float32)
        y = _lrelu(y + b_ref[...])
        o_ref[...] = _ln(y, ln_ref[0:1], ln_ref[1:2])

    return pl.pallas_call(
        body,
        grid=(N // BN,),
        in_specs=[
            pl.BlockSpec((BN, D_IN), lambda i: (i, 0)),
            pl.BlockSpec((D_IN, D), lambda i: (0, 0)),
            pl.BlockSpec((1, D), lambda i: (0, 0)),
            pl.BlockSpec((2, D), lambda i: (0, 0)),
        ],
        out_specs=pl.BlockSpec((BN, D), lambda i: (i, 0)),
        out_shape=jax.ShapeDtypeStruct((N, D), jnp.float32),
        name="tc_encoder",
    )(x, W, b.reshape(1, D), ln)


def _edge(Hh, Ht, Eg, Ws, beu, bmf, bmb, lnr, out_e, name):
    size = Hh.shape[0]
    BE = 640

    def body(hh, ht, eg, ws, b1, b2, b3, lnref, *outs):
        Hhf = hh[...]
        Htf = ht[...]
        Ef = eg[...].astype(jnp.float32)
        Hh_ = Hhf.astype(jnp.bfloat16)
        Ht_ = Htf.astype(jnp.bfloat16)
        Eb = Ef.astype(jnp.bfloat16)
        dot = functools.partial(jnp.dot, preferred_element_type=jnp.float32)
        T = dot(Hh_, ws[0]) + dot(Eb, ws[1]) + dot(Ht_, ws[2]) + b1[...]
        Enf = _ln(_lrelu(T) + Ef, lnref[0:1], lnref[1:2])
        Enb = Enf.astype(jnp.bfloat16)
        Pf = (Hhf * Enf).astype(jnp.bfloat16)
        Pb = (Htf * Enf).astype(jnp.bfloat16)
        mf = dot(Hh_, ws[3]) + dot(Enb, ws[4]) + dot(Pf, ws[5]) + b2[...]
        mb = dot(Ht_, ws[6]) + dot(Enb, ws[7]) + dot(Pb, ws[8]) + b3[...]
        if out_e:
            outs[0][...] = Enb
            outs[1][...] = mf
            outs[2][...] = mb
        else:
            outs[0][...] = mf
            outs[1][...] = mb

    eb = pl.BlockSpec((BE, D), lambda i: (i, 0))

    def cb(shape):
        return pl.BlockSpec(shape, lambda i, _s=shape: tuple(0 for _ in _s))

    in_specs = [eb, eb, eb,
                cb((9, D, D)), cb((1, D)), cb((1, D)), cb((1, D)), cb((2, D))]
    args = [Hh, Ht, Eg, Ws,
            beu.reshape(1, D), bmf.reshape(1, D), bmb.reshape(1, D), lnr]
    out_shape = [jax.ShapeDtypeStruct((size, D), jnp.float32),
                 jax.ShapeDtypeStruct((size, D), jnp.float32)]
    if out_e:
        out_shape = [jax.ShapeDtypeStruct((size, D), jnp.bfloat16)] + out_shape
    return pl.pallas_call(
        body,
        grid=(size // BE,),
        in_specs=in_specs,
        out_specs=[eb] * len(out_shape),
        out_shape=out_shape,
        name=name,
    )(*args)


def _node(aggps, cnt3, H, lnr, name):
    BN = 1000
    na = len(aggps)

    def body(*refs):
        aggs = refs[:na]
        c_ref, h_ref, ln_ref, o_ref = refs[na:]
        a = aggs[0][0] + aggs[0][1]
        for r in aggs[1:]:
            a = a + r[0] + r[1]
        a = a / jnp.maximum(c_ref[...], 1.0)
        x = _lrelu(a) + h_ref[...]
        o_ref[...] = _ln(x, ln_ref[0:1], ln_ref[1:2])

    return pl.pallas_call(
        body,
        grid=(N // BN,),
        in_specs=(
            [pl.BlockSpec((2, BN, D), lambda i: (0, i, 0))] * na
            + [pl.BlockSpec((BN, 1), lambda i: (i, 0)),
               pl.BlockSpec((BN, D), lambda i: (i, 0)),
               pl.BlockSpec((2, D), lambda i: (0, 0))]
        ),
        out_specs=pl.BlockSpec((BN, D), lambda i: (i, 0)),
        out_shape=jax.ShapeDtypeStruct((N, D), jnp.float32),
        name=name,
    )(*aggps, cnt3, H, lnr)


def _sc_gather(tables, idxs, name):
    n = len(tables)
    size = idxs[0].shape[0]
    per_w = size // NW
    chunk = 80
    n_chunks = per_w // chunk
    uniq, tmap = [], []
    for t in tables:
        for u, ut in enumerate(uniq):
            if ut is t:
                tmap.append(u)
                break
        else:
            tmap.append(len(uniq))
            uniq.append(t)
    nu = len(uniq)
    rows_n = [t.shape[0] for t in uniq]
    splits = []
    for rt in rows_n:
        full = -(-rt // NS)
        full += (-full) % 8
        splits.append((full, rt - (NS - 1) * full))
    mesh = plsc.VectorSubcoreMesh(core_axis_name="c", subcore_axis_name="s")
    out_type = tuple(
        jax.ShapeDtypeStruct((size, D), jnp.float32) for _ in range(n))
    scratch = (
        [pltpu.VMEM_SHARED((rt, D), jnp.float32) for rt in rows_n]
        + [pltpu.VMEM((chunk,), jnp.int32) for _ in range(n)]
        + [pltpu.VMEM((chunk, D), jnp.float32) for _ in range(n)]
        + [pltpu.SemaphoreType.DMA]
        + [pltpu.SemaphoreType.DMA for _ in range(n)]
    )

    def body(*refs):
        tbl = refs[:nu]
        idx = refs[nu:nu + n]
        out = refs[nu + n:nu + 2 * n]
        rest = refs[nu + 2 * n:]
        stage = rest[:nu]
        idx_v = rest[nu:nu + n]
        rows_v = rest[nu + n:nu + 2 * n]
        gsem = rest[nu + 2 * n]
        wsem = rest[nu + 2 * n + 1:]
        s = lax.axis_index("s")
        wid = s * NC + lax.axis_index("c")
        base = wid * per_w

        for u in range(nu):
            full, last = splits[u]
            rt0 = pl.multiple_of(s * full, 8)

            @pl.when(s < NS - 1)
            def _(u=u, full=full, rt0=rt0):
                pltpu.sync_copy(tbl[u].at[pl.ds(rt0, full)],
                                stage[u].at[pl.ds(rt0, full)])

            @pl.when(s == NS - 1)
            def _(u=u, last=last, rt0=rt0):
                pltpu.sync_copy(tbl[u].at[pl.ds(rt0, last)],
                                stage[u].at[pl.ds(rt0, last)])
        plsc.subcore_barrier()

        def step(i, carry):
            off = pl.multiple_of(base + i * chunk, 8)
            for t in range(n):
                pltpu.sync_copy(idx[t].at[pl.ds(off, chunk)], idx_v[t])

                @pl.when(i > 0)
                def _(t=t):
                    pltpu.make_async_copy(
                        out[t].at[pl.ds(0, chunk)], rows_v[t], wsem[t]).wait()

                pltpu.async_copy(stage[tmap[t]].at[idx_v[t]], rows_v[t],
                                 gsem).wait()
                pltpu.async_copy(rows_v[t], out[t].at[pl.ds(off, chunk)],
                                 wsem[t])
            return carry

        lax.fori_loop(0, n_chunks, step, 0)
        for t in range(n):
            pltpu.make_async_copy(
                out[t].at[pl.ds(0, chunk)], rows_v[t], wsem[t]).wait()

    f = pl.kernel(body, out_type=out_type, mesh=mesh, scratch_types=scratch,
                  name=name)
    return f(*uniq, *idxs)


def _sc_scatter(mf, mb, tails, heads, with_cnt, name):
    size = tails.shape[0]
    per_w = size // NW
    chunk = 80
    n_chunks = per_w // chunk
    mesh = plsc.VectorSubcoreMesh(core_axis_name="c", subcore_axis_name="s")
    out_type = [jax.ShapeDtypeStruct((NC, N, D), jnp.float32)]
    scratch = [
        pltpu.VMEM_SHARED((N, D), jnp.float32),
        pltpu.VMEM((chunk, D), jnp.float32),
        pltpu.VMEM((chunk, D), jnp.float32),
        pltpu.VMEM((chunk,), jnp.int32),
        pltpu.VMEM((chunk,), jnp.int32),
    ] + [pltpu.SemaphoreType.DMA] * 6
    zeros_blk = jnp.zeros((_WFULL, D), jnp.float32)
    args = [mf, mb, tails, heads, zeros_blk]
    if with_cnt:
        out_type.append(jax.ShapeDtypeStruct((NC * N,), jnp.float32))
        scratch += [pltpu.VMEM_SHARED((N,), jnp.float32),
                    pltpu.VMEM((chunk,), jnp.float32),
                    pltpu.VMEM((N,), jnp.float32),
                    pltpu.SemaphoreType.DMA, pltpu.SemaphoreType.DMA]
        args += [jnp.zeros((N,), jnp.float32), jnp.ones((chunk,), jnp.float32)]

    def body(*refs):
        if with_cnt:
            (mf_h, mb_h, t_h, h_h, zb_h, zn_h, on_h, agg_o, cnt_o,
             agg_s, b0, b1, i0, i1, lsi0, lsb0, lsi1, lsb1, ss0, ss1,
             cnt_s, ones_v, cnt_v, cs0, cs1) = refs
        else:
            (mf_h, mb_h, t_h, h_h, zb_h, agg_o,
             agg_s, b0, b1, i0, i1, lsi0, lsb0, lsi1, lsb1, ss0, ss1) = refs
        c = lax.axis_index("c")
        s = lax.axis_index("s")
        wid = s * NC + c
        r0 = pl.multiple_of(s * _WFULL, 8)

        @pl.when(s < NS - 1)
        def _():
            pltpu.sync_copy(zb_h, agg_s.at[pl.ds(r0, _WFULL)])

        @pl.when(s == NS - 1)
        def _():
            pltpu.sync_copy(zb_h.at[pl.ds(0, _WLAST)],
                            agg_s.at[pl.ds(r0, _WLAST)])

        if with_cnt:
            @pl.when(s == 0)
            def _():
                pltpu.sync_copy(zn_h, cnt_v)
                pltpu.sync_copy(cnt_v, cnt_s)
            pltpu.sync_copy(on_h, ones_v)
        plsc.subcore_barrier()
        base = wid * per_w

        pltpu.async_copy(t_h.at[pl.ds(base, chunk)], i0, lsi0)
        pltpu.async_copy(mf_h.at[pl.ds(base, chunk)], b0, lsb0)
        pltpu.async_copy(h_h.at[pl.ds(base, chunk)], i1, lsi1)
        pltpu.async_copy(mb_h.at[pl.ds(base, chunk)], b1, lsb1)

        def step(j, carry):
            off_n = pl.multiple_of(base + (j + 1) * chunk, 8)
            pltpu.make_async_copy(t_h.at[pl.ds(0, chunk)], i0, lsi0).wait()
            pltpu.make_async_copy(mf_h.at[pl.ds(0, chunk)], b0, lsb0).wait()
            d0 = pltpu.async_copy(b0, agg_s.at[i0], ss0, add=True)
            if with_cnt:
                dc0 = pltpu.async_copy(ones_v, cnt_s.at[i0], cs0, add=True)
            pltpu.make_async_copy(h_h.at[pl.ds(0, chunk)], i1, lsi1).wait()
            pltpu.make_async_copy(mb_h.at[pl.ds(0, chunk)], b1, lsb1).wait()
            d1 = pltpu.async_copy(b1, agg_s.at[i1], ss1, add=True)
            if with_cnt:
                dc1 = pltpu.async_copy(ones_v, cnt_s.at[i1], cs1, add=True)
            d0.wait()
            if with_cnt:
                dc0.wait()

            @pl.when(j < n_chunks - 1)
            def _():
                pltpu.async_copy(t_h.at[pl.ds(off_n, chunk)], i0, lsi0)
                pltpu.async_copy(mf_h.at[pl.ds(off_n, chunk)], b0, lsb0)

            d1.wait()
            if with_cnt:
                dc1.wait()

            @pl.when(j < n_chunks - 1)
            def _():
                pltpu.async_copy(h_h.at[pl.ds(off_n, chunk)], i1, lsi1)
                pltpu.async_copy(mb_h.at[pl.ds(off_n, chunk)], b1, lsb1)
            return carry

        lax.fori_loop(0, n_chunks, step, 0)
        plsc.subcore_barrier()

        @pl.when(s < NS - 1)
        def _():
            pltpu.sync_copy(agg_s.at[pl.ds(r0, _WFULL)],
                            agg_o.at[c, pl.ds(r0, _WFULL)])

        @pl.when(s == NS - 1)
        def _():
            pltpu.sync_copy(agg_s.at[pl.ds(r0, _WLAST)],
                            agg_o.at[c, pl.ds(r0, _WLAST)])

        if with_cnt:
            @pl.when(s == 0)
            def _():
                pltpu.sync_copy(cnt_s, cnt_v)
                pltpu.sync_copy(cnt_v,
                                cnt_o.at[pl.ds(pl.multiple_of(c * N, 8), N)])

    f = pl.kernel(body, out_type=tuple(out_type), mesh=mesh,
                  scratch_types=scratch, name=name)
    res = f(*args)
    return res if with_cnt else res[0]


def _layer_weights(W_eu, b_eu, W_mf, b_mf, W_mb, b_mb, ln_eu, l):
    Wh, We, Wt = W_eu[l, :D], W_eu[l, D:2 * D], W_eu[l, 2 * D:]
    Afh = W_mf[l, :D] + W_mf[l, 2 * D:3 * D]
    Afe = W_mf[l, D:2 * D] + W_mf[l, 2 * D:3 * D]
    Afp = W_mf[l, 3 * D:]
    Abt = W_mb[l, :D] + W_mb[l, 2 * D:3 * D]
    Abe = W_mb[l, D:2 * D] + W_mb[l, 2 * D:3 * D]
    Abp = W_mb[l, 3 * D:]
    Ws = jnp.stack([Wh, We, Wt, Afh, Afe, Afp, Abt, Abe, Abp])
    return Ws.astype(jnp.bfloat16), b_eu[l], b_mf[l], b_mb[l], ln_eu[l]


def kernel(entity_feat, ht, r_tensor, r_relative, W_ent, b_ent, ln_ent,
           rel_emb, dir_emb, W_eu, b_eu, ln_eu, W_mf, b_mf, W_mb, b_mb,
           ln_mp):
    heads = ht[:, 0]
    tails = ht[:, 1]
    T2 = (rel_emb[:, None, :] + dir_emb[None, :, :]).reshape(-1, D)
    idx_e = r_tensor * 2 + r_relative
    bounds = [0]
    for sz in _SIZES:
        bounds.append(bounds[-1] + sz)
    halves = tuple((bounds[i], bounds[i + 1]) for i in range(NSPLIT))

    H = _encoder(entity_feat, W_ent, b_ent, ln_ent)
    gh = [_sc_gather([H, H, T2],
                     [heads[lo:hi], tails[lo:hi], idx_e[lo:hi]],
                     f"sc_gH0_{i}")
          for i, (lo, hi) in enumerate(halves)]
    Eh = [g[2] for g in gh]

    cnt3 = None
    for l in range(L):
        Ws, beu, bmf, bmb, lnr = _layer_weights(
            W_eu, b_eu, W_mf, b_mf, W_mb, b_mb, ln_eu, l)
        first = l == 0
        eouts = [_edge(gh[i][0], gh[i][1], Eh[i], Ws, beu, bmf, bmb, lnr,
                       out_e=first, name=f"tc_edge{l}_{i}")
                 for i in range(NSPLIT)]
        if first:
            Eh = [eo[0] for eo in eouts]
            scs = [_sc_scatter(eo[1], eo[2], tails[lo:hi], heads[lo:hi],
                               True, f"sc_scat{l}_{i}")
                   for i, (eo, (lo, hi)) in enumerate(zip(eouts, halves))]
            aggps = [s[0] for s in scs]
            cnt_flat = scs[0][1]
            for s_ in scs[1:]:
                cnt_flat = cnt_flat + s_[1]
            cnt3 = (cnt_flat[:N] + cnt_flat[N:]).reshape(N, 1)
        else:
            scs = [_sc_scatter(eo[0], eo[1], tails[lo:hi], heads[lo:hi],
                               False, f"sc_scat{l}_{i}")
                   for i, (eo, (lo, hi)) in enumerate(zip(eouts, halves))]
            aggps = [s for s in scs]
        H = _node(aggps, cnt3, H, ln_mp[l], f"tc_node{l}")
        if first:
            gh = [_sc_gather([H, H], [heads[lo:hi], tails[lo:hi]],
                             f"sc_gH1_{i}")
                  for i, (lo, hi) in enumerate(halves)]
    return H


# speedup vs baseline: 1.7152x; 1.0072x over previous
import functools

import jax
import jax.numpy as jnp
from jax import lax
from jax.experimental import pallas as pl
from jax.experimental.pallas import tpu as pltpu
from jax.experimental.pallas import tpu_sc as plsc

N = 10000
M = 320000
D_IN = 768
D = 128
L = 2

NC = 2
NS = 16
NW = NC * NS
ROWS_PER_TILE = N // NS
_WFULL = 632
_WLAST = N - (NS - 1) * _WFULL
_SIZES = (25600, 64000, 64000, 64000, 64000, 38400)
NSPLIT = len(_SIZES)


def _lrelu(x):
    return jnp.where(x >= 0, x, 0.01 * x)


def _ln(x, g, b):
    mu = jnp.mean(x, axis=-1, keepdims=True)
    var = jnp.mean((x - mu) * (x - mu), axis=-1, keepdims=True)
    return (x - mu) * lax.rsqrt(var + 1e-5) * g + b


def _chunk_for(per_w):
    for c in (400, 200, 100, 40):
        if per_w % c == 0 and c % 8 == 0:
            return c
    raise ValueError(per_w)


def _encoder(x, W, b, ln):
    BN = 1000

    def body(x_ref, w_ref, b_ref, ln_ref, o_ref):
        y = jnp.dot(x_ref[...], w_ref[...], preferred_element_type=jnp.float32)
        y = _lrelu(y + b_ref[...])
        o_ref[...] = _ln(y, ln_ref[0:1], ln_ref[1:2])

    return pl.pallas_call(
        body,
        grid=(N // BN,),
        in_specs=[
            pl.BlockSpec((BN, D_IN), lambda i: (i, 0)),
            pl.BlockSpec((D_IN, D), lambda i: (0, 0)),
            pl.BlockSpec((1, D), lambda i: (0, 0)),
            pl.BlockSpec((2, D), lambda i: (0, 0)),
        ],
        out_specs=pl.BlockSpec((BN, D), lambda i: (i, 0)),
        out_shape=jax.ShapeDtypeStruct((N, D), jnp.float32),
        name="tc_encoder",
    )(x, W, b.reshape(1, D), ln)


def _edge(Hh, Ht, Eg, Ws, beu, bmf, bmb, lnr, out_e, name):
    size = Hh.shape[0]
    BE = 640

    def body(hh, ht, eg, ws, b1, b2, b3, lnref, *outs):
        Hhf = hh[...]
        Htf = ht[...]
        Ef = eg[...].astype(jnp.float32)
        Hh_ = Hhf.astype(jnp.bfloat16)
        Ht_ = Htf.astype(jnp.bfloat16)
        Eb = Ef.astype(jnp.bfloat16)
        dot = functools.partial(jnp.dot, preferred_element_type=jnp.float32)
        T = dot(Hh_, ws[0]) + dot(Eb, ws[1]) + dot(Ht_, ws[2]) + b1[...]
        Enf = _ln(_lrelu(T) + Ef, lnref[0:1], lnref[1:2])
        Enb = Enf.astype(jnp.bfloat16)
        Pf = (Hhf * Enf).astype(jnp.bfloat16)
        Pb = (Htf * Enf).astype(jnp.bfloat16)
        mf = dot(Hh_, ws[3]) + dot(Enb, ws[4]) + dot(Pf, ws[5]) + b2[...]
        mb = dot(Ht_, ws[6]) + dot(Enb, ws[7]) + dot(Pb, ws[8]) + b3[...]
        if out_e:
            outs[0][...] = Enb
            outs[1][...] = mf
            outs[2][...] = mb
        else:
            outs[0][...] = mf
            outs[1][...] = mb

    eb = pl.BlockSpec((BE, D), lambda i: (i, 0))

    def cb(shape):
        return pl.BlockSpec(shape, lambda i, _s=shape: tuple(0 for _ in _s))

    in_specs = [eb, eb, eb,
                cb((9, D, D)), cb((1, D)), cb((1, D)), cb((1, D)), cb((2, D))]
    args = [Hh, Ht, Eg, Ws,
            beu.reshape(1, D), bmf.reshape(1, D), bmb.reshape(1, D), lnr]
    out_shape = [jax.ShapeDtypeStruct((size, D), jnp.float32),
                 jax.ShapeDtypeStruct((size, D), jnp.float32)]
    if out_e:
        out_shape = [jax.ShapeDtypeStruct((size, D), jnp.bfloat16)] + out_shape
    return pl.pallas_call(
        body,
        grid=(size // BE,),
        in_specs=in_specs,
        out_specs=[eb] * len(out_shape),
        out_shape=out_shape,
        name=name,
    )(*args)


def _node(aggps, cnt3, H, lnr, name):
    BN = 1000
    na = len(aggps)

    def body(*refs):
        aggs = refs[:na]
        c_ref, h_ref, ln_ref, o_ref = refs[na:]
        a = aggs[0][0] + aggs[0][1]
        for r in aggs[1:]:
            a = a + r[0] + r[1]
        a = a / jnp.maximum(c_ref[...], 1.0)
        x = _lrelu(a) + h_ref[...]
        o_ref[...] = _ln(x, ln_ref[0:1], ln_ref[1:2])

    return pl.pallas_call(
        body,
        grid=(N // BN,),
        in_specs=(
            [pl.BlockSpec((2, BN, D), lambda i: (0, i, 0))] * na
            + [pl.BlockSpec((BN, 1), lambda i: (i, 0)),
               pl.BlockSpec((BN, D), lambda i: (i, 0)),
               pl.BlockSpec((2, D), lambda i: (0, 0))]
        ),
        out_specs=pl.BlockSpec((BN, D), lambda i: (i, 0)),
        out_shape=jax.ShapeDtypeStruct((N, D), jnp.float32),
        name=name,
    )(*aggps, cnt3, H, lnr)


def _sc_gather(tables, idxs, name):
    n = len(tables)
    size = idxs[0].shape[0]
    per_w = size // NW
    chunk = 80
    n_chunks = per_w // chunk
    uniq, tmap = [], []
    for t in tables:
        for u, ut in enumerate(uniq):
            if ut is t:
                tmap.append(u)
                break
        else:
            tmap.append(len(uniq))
            uniq.append(t)
    nu = len(uniq)
    rows_n = [t.shape[0] for t in uniq]
    splits = []
    for rt in rows_n:
        full = -(-rt // NS)
        full += (-full) % 8
        splits.append((full, rt - (NS - 1) * full))
    mesh = plsc.VectorSubcoreMesh(core_axis_name="c", subcore_axis_name="s")
    out_type = tuple(
        jax.ShapeDtypeStruct((size, D), jnp.float32) for _ in range(n))
    scratch = (
        [pltpu.VMEM_SHARED((rt, D), jnp.float32) for rt in rows_n]
        + [pltpu.VMEM((chunk,), jnp.int32) for _ in range(n)]
        + [pltpu.VMEM((chunk, D), jnp.float32) for _ in range(n)]
        + [pltpu.SemaphoreType.DMA]
        + [pltpu.SemaphoreType.DMA for _ in range(n)]
    )

    def body(*refs):
        tbl = refs[:nu]
        idx = refs[nu:nu + n]
        out = refs[nu + n:nu + 2 * n]
        rest = refs[nu + 2 * n:]
        stage = rest[:nu]
        idx_v = rest[nu:nu + n]
        rows_v = rest[nu + n:nu + 2 * n]
        gsem = rest[nu + 2 * n]
        wsem = rest[nu + 2 * n + 1:]
        s = lax.axis_index("s")
        wid = s * NC + lax.axis_index("c")
        base = wid * per_w

        for u in range(nu):
            full, last = splits[u]
            rt0 = pl.multiple_of(s * full, 8)

            @pl.when(s < NS - 1)
            def _(u=u, full=full, rt0=rt0):
                pltpu.sync_copy(tbl[u].at[pl.ds(rt0, full)],
                                stage[u].at[pl.ds(rt0, full)])

            @pl.when(s == NS - 1)
            def _(u=u, last=last, rt0=rt0):
                pltpu.sync_copy(tbl[u].at[pl.ds(rt0, last)],
                                stage[u].at[pl.ds(rt0, last)])
        plsc.subcore_barrier()

        def step(i, carry):
            off = pl.multiple_of(base + i * chunk, 8)
            for t in range(n):
                pltpu.sync_copy(idx[t].at[pl.ds(off, chunk)], idx_v[t])

                @pl.when(i > 0)
                def _(t=t):
                    pltpu.make_async_copy(
                        out[t].at[pl.ds(0, chunk)], rows_v[t], wsem[t]).wait()

                pltpu.async_copy(stage[tmap[t]].at[idx_v[t]], rows_v[t],
                                 gsem).wait()
                pltpu.async_copy(rows_v[t], out[t].at[pl.ds(off, chunk)],
                                 wsem[t])
            return carry

        lax.fori_loop(0, n_chunks, step, 0)
        for t in range(n):
            pltpu.make_async_copy(
                out[t].at[pl.ds(0, chunk)], rows_v[t], wsem[t]).wait()

    f = pl.kernel(body, out_type=out_type, mesh=mesh, scratch_types=scratch,
                  name=name)
    return f(*uniq, *idxs)


def _sc_scatter(mf, mb, tails, heads, with_cnt, name):
    size = tails.shape[0]
    per_w = size // NW
    chunk = 80
    n_chunks = per_w // chunk
    mesh = plsc.VectorSubcoreMesh(core_axis_name="c", subcore_axis_name="s")
    out_type = [jax.ShapeDtypeStruct((NC, N, D), jnp.float32)]
    scratch = [
        pltpu.VMEM_SHARED((N, D), jnp.float32),
        pltpu.VMEM((chunk, D), jnp.float32),
        pltpu.VMEM((chunk, D), jnp.float32),
        pltpu.VMEM((chunk,), jnp.int32),
        pltpu.VMEM((chunk,), jnp.int32),
    ] + [pltpu.SemaphoreType.DMA] * 6
    zeros_blk = jnp.zeros((_WFULL, D), jnp.float32)
    args = [mf, mb, tails, heads, zeros_blk]
    if with_cnt:
        out_type.append(jax.ShapeDtypeStruct((NC * N,), jnp.float32))
        scratch += [pltpu.VMEM_SHARED((N,), jnp.float32),
                    pltpu.VMEM((chunk,), jnp.float32),
                    pltpu.VMEM((N,), jnp.float32),
                    pltpu.SemaphoreType.DMA, pltpu.SemaphoreType.DMA]
        args += [jnp.zeros((N,), jnp.float32), jnp.ones((chunk,), jnp.float32)]

    def body(*refs):
        if with_cnt:
            (mf_h, mb_h, t_h, h_h, zb_h, zn_h, on_h, agg_o, cnt_o,
             agg_s, b0, b1, i0, i1, lsi0, lsb0, lsi1, lsb1, ss0, ss1,
             cnt_s, ones_v, cnt_v, cs0, cs1) = refs
        else:
            (mf_h, mb_h, t_h, h_h, zb_h, agg_o,
             agg_s, b0, b1, i0, i1, lsi0, lsb0, lsi1, lsb1, ss0, ss1) = refs
        c = lax.axis_index("c")
        s = lax.axis_index("s")
        wid = s * NC + c
        r0 = pl.multiple_of(s * _WFULL, 8)

        @pl.when(s < NS - 1)
        def _():
            pltpu.sync_copy(zb_h, agg_s.at[pl.ds(r0, _WFULL)])

        @pl.when(s == NS - 1)
        def _():
            pltpu.sync_copy(zb_h.at[pl.ds(0, _WLAST)],
                            agg_s.at[pl.ds(r0, _WLAST)])

        if with_cnt:
            @pl.when(s == 0)
            def _():
                pltpu.sync_copy(zn_h, cnt_v)
                pltpu.sync_copy(cnt_v, cnt_s)
            pltpu.sync_copy(on_h, ones_v)
        plsc.subcore_barrier()
        base = wid * per_w

        pltpu.async_copy(t_h.at[pl.ds(base, chunk)], i0, lsi0)
        pltpu.async_copy(mf_h.at[pl.ds(base, chunk)], b0, lsb0)
        pltpu.async_copy(h_h.at[pl.ds(base, chunk)], i1, lsi1)
        pltpu.async_copy(mb_h.at[pl.ds(base, chunk)], b1, lsb1)

        def step(j, carry):
            off_n = pl.multiple_of(base + (j + 1) * chunk, 8)
            pltpu.make_async_copy(t_h.at[pl.ds(0, chunk)], i0, lsi0).wait()
            pltpu.make_async_copy(mf_h.at[pl.ds(0, chunk)], b0, lsb0).wait()
            d0 = pltpu.async_copy(b0, agg_s.at[i0], ss0, add=True)
            if with_cnt:
                dc0 = pltpu.async_copy(ones_v, cnt_s.at[i0], cs0, add=True)
            pltpu.make_async_copy(h_h.at[pl.ds(0, chunk)], i1, lsi1).wait()
            pltpu.make_async_copy(mb_h.at[pl.ds(0, chunk)], b1, lsb1).wait()
            d1 = pltpu.async_copy(b1, agg_s.at[i1], ss1, add=True)
            if with_cnt:
                dc1 = pltpu.async_copy(ones_v, cnt_s.at[i1], cs1, add=True)
            d0.wait()
            if with_cnt:
                dc0.wait()

            @pl.when(j < n_chunks - 1)
            def _():
                pltpu.async_copy(t_h.at[pl.ds(off_n, chunk)], i0, lsi0)
                pltpu.async_copy(mf_h.at[pl.ds(off_n, chunk)], b0, lsb0)

            d1.wait()
            if with_cnt:
                dc1.wait()

            @pl.when(j < n_chunks - 1)
            def _():
                pltpu.async_copy(h_h.at[pl.ds(off_n, chunk)], i1, lsi1)
                pltpu.async_copy(mb_h.at[pl.ds(off_n, chunk)], b1, lsb1)
            return carry

        lax.fori_loop(0, n_chunks, step, 0)
        plsc.subcore_barrier()

        @pl.when(s < NS - 1)
        def _():
            pltpu.sync_copy(agg_s.at[pl.ds(r0, _WFULL)],
                            agg_o.at[c, pl.ds(r0, _WFULL)])

        @pl.when(s == NS - 1)
        def _():
            pltpu.sync_copy(agg_s.at[pl.ds(r0, _WLAST)],
                            agg_o.at[c, pl.ds(r0, _WLAST)])

        if with_cnt:
            @pl.when(s == 0)
            def _():
                pltpu.sync_copy(cnt_s, cnt_v)
                pltpu.sync_copy(cnt_v,
                                cnt_o.at[pl.ds(pl.multiple_of(c * N, 8), N)])

    f = pl.kernel(body, out_type=tuple(out_type), mesh=mesh,
                  scratch_types=scratch, name=name)
    res = f(*args)
    return res if with_cnt else res[0]


def _layer_weights(W_eu, b_eu, W_mf, b_mf, W_mb, b_mb, ln_eu, l):
    Wh, We, Wt = W_eu[l, :D], W_eu[l, D:2 * D], W_eu[l, 2 * D:]
    Afh = W_mf[l, :D] + W_mf[l, 2 * D:3 * D]
    Afe = W_mf[l, D:2 * D] + W_mf[l, 2 * D:3 * D]
    Afp = W_mf[l, 3 * D:]
    Abt = W_mb[l, :D] + W_mb[l, 2 * D:3 * D]
    Abe = W_mb[l, D:2 * D] + W_mb[l, 2 * D:3 * D]
    Abp = W_mb[l, 3 * D:]
    Ws = jnp.stack([Wh, We, Wt, Afh, Afe, Afp, Abt, Abe, Abp])
    return Ws.astype(jnp.bfloat16), b_eu[l], b_mf[l], b_mb[l], ln_eu[l]


def kernel(entity_feat, ht, r_tensor, r_relative, W_ent, b_ent, ln_ent,
           rel_emb, dir_emb, W_eu, b_eu, ln_eu, W_mf, b_mf, W_mb, b_mb,
           ln_mp):
    heads = ht[:, 0]
    tails = ht[:, 1]
    T2 = (rel_emb[:, None, :] + dir_emb[None, :, :]).reshape(-1, D)
    idx_e = r_tensor * 2 + r_relative
    bounds = [0]
    for sz in _SIZES:
        bounds.append(bounds[-1] + sz)
    halves = tuple((bounds[i], bounds[i + 1]) for i in range(NSPLIT))

    H = _encoder(entity_feat, W_ent, b_ent, ln_ent)
    gh = [_sc_gather([H, H, T2],
                     [heads[lo:hi], tails[lo:hi], idx_e[lo:hi]],
                     f"sc_gH0_{i}")
          for i, (lo, hi) in enumerate(halves)]
    Eh = [g[2] for g in gh]

    cnt3 = None
    for l in range(L):
        Ws, beu, bmf, bmb, lnr = _layer_weights(
            W_eu, b_eu, W_mf, b_mf, W_mb, b_mb, ln_eu, l)
        first = l == 0
        eouts = [_edge(gh[i][0], gh[i][1], Eh[i], Ws, beu, bmf, bmb, lnr,
                       out_e=first, name=f"tc_edge{l}_{i}")
                 for i in range(NSPLIT)]
        if first:
            Eh = [eo[0] for eo in eouts]
            scs = [_sc_scatter(eo[1], eo[2], tails[lo:hi], heads[lo:hi],
                               True, f"sc_scat{l}_{i}")
                   for i, (eo, (lo, hi)) in enumerate(zip(eouts, halves))]
            aggps = [s[0] for s in scs]
            cnt_flat = scs[0][1]
            for s_ in scs[1:]:
                cnt_flat = cnt_flat + s_[1]
            cnt3 = (cnt_flat[:N] + cnt_flat[N:]).reshape(N, 1)
        else:
            scs = [_sc_scatter(eo[0], eo[1], tails[lo:hi], heads[lo:hi],
                               False, f"sc_scat{l}_{i}")
                   for i, (eo, (lo, hi)) in enumerate(zip(eouts, halves))]
            aggps = [s for s in scs]
        H = _node(aggps, cnt3, H, ln_mp[l], f"tc_node{l}")
        if first:
            gh = [_sc_gather([H, H], [heads[lo:hi], tails[lo:hi]],
                             f"sc_gH1_{i}")
                  for i, (lo, hi) in enumerate(halves)]
    return H
